# baseline (speedup 1.0000x reference)
import functools

import jax
import jax.numpy as jnp
from jax import lax
from jax.experimental import pallas as pl
from jax.experimental.pallas import tpu as pltpu
from jax.experimental.pallas import tpu_sc as plsc

NC = 2
NS = 16
L = 16
B = 8
N = 2048
M = 2048
WPB = 4
CHUNK = N // WPB

_mesh = plsc.VectorSubcoreMesh(core_axis_name="c", subcore_axis_name="s", num_cores=NC, num_subcores=NS)


@functools.partial(
    pl.kernel,
    out_type=(
        jax.ShapeDtypeStruct((B, N), jnp.float32),
        jax.ShapeDtypeStruct((B, M), jnp.float32),
        jax.ShapeDtypeStruct((B, N), jnp.int32),
        jax.ShapeDtypeStruct((B, M), jnp.int32),
    ),
    mesh=_mesh,
    compiler_params=pltpu.CompilerParams(needs_layout_passes=False),
    scratch_types=dict(
        x1v=pltpu.VMEM((CHUNK * 3 + L,), jnp.float32),
        x1r=pltpu.VMEM((CHUNK * 3 + L,), jnp.float32),
        x2v=pltpu.VMEM((3 * M,), jnp.float32),
        s2v=pltpu.VMEM((M,), jnp.float32),
        rminv=pltpu.VMEM((CHUNK,), jnp.float32),
        ridxv=pltpu.VMEM((CHUNK,), jnp.int32),
        cminv=pltpu.VMEM((M,), jnp.float32),
        cidxv=pltpu.VMEM((M,), jnp.int32),
        mmin=pltpu.VMEM((WPB * M,), jnp.float32),
        midx=pltpu.VMEM((WPB * M,), jnp.int32),
        shmin=pltpu.VMEM_SHARED((NS * M,), jnp.float32),
        shidx=pltpu.VMEM_SHARED((NS * M,), jnp.int32),
    ),
)
def _chamfer_sc(x1_hbm, x2_hbm, d1_hbm, d2_hbm, i1_hbm, i2_hbm,
                x1v, x1r, x2v, s2v, rminv, ridxv, cminv, cidxv,
                mmin, midx, shmin, shidx):
  c = lax.axis_index("c")
  s = lax.axis_index("s")
  wid = c * NS + s
  b = wid // WPB
  chunk = wid % WPB
  row0 = chunk * CHUNK

  pltpu.sync_copy(x1_hbm.at[b, pl.ds(row0 * 3, CHUNK * 3)],
                  x1v.at[pl.ds(0, CHUNK * 3)])
  pltpu.sync_copy(x2_hbm.at[b], x2v)

  lanes = lax.iota(jnp.int32, L)
  inf16 = jnp.full((L,), jnp.inf, jnp.float32)
  zero16 = jnp.zeros((L,), jnp.int32)

  def _bf16r(v):
    u = plsc.bitcast(v, jnp.uint32)
    u = (u + jnp.uint32(0x7FFF) + ((u >> jnp.uint32(16)) & jnp.uint32(1)))
    u = u & jnp.uint32(0xFFFF0000)
    return plsc.bitcast(u, jnp.float32)

  def _prep(jc, _):
    off = jc * L
    b0 = x2v[pl.ds(off, L)]
    b1 = x2v[pl.ds(M + off, L)]
    b2 = x2v[pl.ds(2 * M + off, L)]
    s2v[pl.ds(off, L)] = (b0 * b0 + b1 * b1) + b2 * b2
    x2v[pl.ds(off, L)] = _bf16r(b0)
    x2v[pl.ds(M + off, L)] = _bf16r(b1)
    x2v[pl.ds(2 * M + off, L)] = _bf16r(b2)
    cminv[pl.ds(off, L)] = inf16
    cidxv[pl.ds(off, L)] = zero16
    return 0
  lax.fori_loop(0, M // L, _prep, 0, unroll=4)

  def _prep1(jc, _):
    off = jc * L
    x1r[pl.ds(off, L)] = _bf16r(x1v[pl.ds(off, L)])
    return 0
  lax.fori_loop(0, (CHUNK * 3 + L) // L, _prep1, 0, unroll=4)

  def _row(r, carry):
    accm, acci, g = carry
    i = g * L + r
    va = x1v[pl.ds(3 * i, L)]
    A0 = jnp.full((L,), va[0], jnp.float32)
    A1 = jnp.full((L,), va[1], jnp.float32)
    A2 = jnp.full((L,), va[2], jnp.float32)
    s1r = (A0 * A0 + A1 * A1) + A2 * A2
    vb = x1r[pl.ds(3 * i, L)]
    a0 = jnp.full((L,), vb[0], jnp.float32)
    a1 = jnp.full((L,), vb[1], jnp.float32)
    a2 = jnp.full((L,), vb[2], jnp.float32)
    iv = jnp.full((L,), row0 + i, jnp.int32)

    def _col(jc, carry):
      rmin, ridx = carry
      off = jc * L
      b0 = x2v[pl.ds(off, L)]
      b1 = x2v[pl.ds(M + off, L)]
      b2 = x2v[pl.ds(2 * M + off, L)]
      s2c = s2v[pl.ds(off, L)]
      cross = (a0 * b0 + a1 * b1) + a2 * b2
      d = (s1r + s2c) - 2.0 * cross
      jv = lanes + off
      mr = d < rmin
      rmin = jnp.where(mr, d, rmin)
      ridx = jnp.where(mr, jv, ridx)
      cmin = cminv[pl.ds(off, L)]
      cidx = cidxv[pl.ds(off, L)]
      mc = d < cmin
      cminv[pl.ds(off, L)] = jnp.where(mc, d, cmin)
      cidxv[pl.ds(off, L)] = jnp.where(mc, iv, cidx)
      return rmin, ridx

    rmin, ridx = lax.fori_loop(0, M // L, _col, (inf16, zero16), unroll=8)
    rs = jnp.min(rmin)
    ri = jnp.min(jnp.where(rmin == rs, ridx, jnp.int32(M)))
    lm = lanes == r
    accm = jnp.where(lm, rs, accm)
    acci = jnp.where(lm, ri, acci)
    return accm, acci, g

  def _rowgrp(g, _):
    accm, acci, _g = lax.fori_loop(0, L, _row, (inf16, zero16, g))
    rminv[pl.ds(g * L, L)] = accm
    ridxv[pl.ds(g * L, L)] = acci
    return 0
  lax.fori_loop(0, CHUNK // L, _rowgrp, 0)

  pltpu.sync_copy(rminv, d1_hbm.at[b, pl.ds(row0, CHUNK)])
  pltpu.sync_copy(ridxv, i1_hbm.at[b, pl.ds(row0, CHUNK)])

  pltpu.sync_copy(cminv, shmin.at[pl.ds(s * M, M)])
  pltpu.sync_copy(cidxv, shidx.at[pl.ds(s * M, M)])
  plsc.subcore_barrier()

  @pl.when(chunk == 0)
  def _merge():
    pltpu.sync_copy(shmin.at[pl.ds(s * M, WPB * M)], mmin)
    pltpu.sync_copy(shidx.at[pl.ds(s * M, WPB * M)], midx)

    def _mrg(jc, _):
      off = jc * L
      m = mmin[pl.ds(off, L)]
      ix = midx[pl.ds(off, L)]
      for k in range(1, WPB):
        mk = mmin[pl.ds(k * M + off, L)]
        ik = midx[pl.ds(k * M + off, L)]
        lt = mk < m
        m = jnp.where(lt, mk, m)
        ix = jnp.where(lt, ik, ix)
      cminv[pl.ds(off, L)] = m
      cidxv[pl.ds(off, L)] = ix
      return 0
    lax.fori_loop(0, M // L, _mrg, 0)
    pltpu.sync_copy(cminv, d2_hbm.at[b])
    pltpu.sync_copy(cidxv, i2_hbm.at[b])


@jax.jit
def kernel(input1, input2):
  x1f = input1.reshape(B, N * 3)
  x2t = jnp.swapaxes(input2, 1, 2).reshape(B, 3 * M)
  d1, d2, i1, i2 = _chamfer_sc(x1f, x2t)
  return d1, d2, i1, i2


# speedup vs baseline: 2.9489x; 2.9489x over previous
import functools

import jax
import jax.numpy as jnp
from jax import lax
from jax.experimental import pallas as pl
from jax.experimental.pallas import tpu as pltpu
from jax.experimental.pallas import tpu_sc as plsc

NC = 2
NS = 16
L = 16
B = 8
N = 2048
M = 2048
WPB = 4
CHUNK = N // WPB

_mesh = plsc.VectorSubcoreMesh(core_axis_name="c", subcore_axis_name="s", num_cores=NC, num_subcores=NS)


@functools.partial(
    pl.kernel,
    out_type=(
        jax.ShapeDtypeStruct((B, N), jnp.float32),
        jax.ShapeDtypeStruct((B, M), jnp.float32),
        jax.ShapeDtypeStruct((B, N), jnp.int32),
        jax.ShapeDtypeStruct((B, M), jnp.int32),
    ),
    mesh=_mesh,
    compiler_params=pltpu.CompilerParams(needs_layout_passes=False),
    scratch_types=dict(
        x1v=pltpu.VMEM((CHUNK * 3 + L,), jnp.float32),
        x1r=pltpu.VMEM((CHUNK * 3 + L,), jnp.float32),
        x2v=pltpu.VMEM((3 * M,), jnp.float32),
        s2v=pltpu.VMEM((M,), jnp.float32),
        rminv=pltpu.VMEM((CHUNK,), jnp.float32),
        ridxv=pltpu.VMEM((CHUNK,), jnp.int32),
        cminv=pltpu.VMEM((M,), jnp.float32),
        cidxv=pltpu.VMEM((M,), jnp.int32),
        mmin=pltpu.VMEM((WPB * M,), jnp.float32),
        midx=pltpu.VMEM((WPB * M,), jnp.int32),
        shmin=pltpu.VMEM_SHARED((NS * M,), jnp.float32),
        shidx=pltpu.VMEM_SHARED((NS * M,), jnp.int32),
    ),
)
def _chamfer_sc(x1_hbm, x2_hbm, d1_hbm, d2_hbm, i1_hbm, i2_hbm,
                x1v, x1r, x2v, s2v, rminv, ridxv, cminv, cidxv,
                mmin, midx, shmin, shidx):
  c = lax.axis_index("c")
  s = lax.axis_index("s")
  wid = c * NS + s
  b = wid // WPB
  chunk = wid % WPB
  row0 = chunk * CHUNK

  pltpu.sync_copy(x1_hbm.at[b, pl.ds(row0 * 3, CHUNK * 3)],
                  x1v.at[pl.ds(0, CHUNK * 3)])
  pltpu.sync_copy(x2_hbm.at[b], x2v)

  lanes = lax.iota(jnp.int32, L)
  inf16 = jnp.full((L,), jnp.inf, jnp.float32)
  zero16 = jnp.zeros((L,), jnp.int32)

  def _bf16r(v):
    u = plsc.bitcast(v, jnp.uint32)
    u = (u + jnp.uint32(0x7FFF) + ((u >> jnp.uint32(16)) & jnp.uint32(1)))
    u = u & jnp.uint32(0xFFFF0000)
    return plsc.bitcast(u, jnp.float32)

  @plsc.parallel_loop(0, M // L, unroll=4)
  def _prep(jc):
    off = jc * L
    b0 = x2v[pl.ds(off, L)]
    b1 = x2v[pl.ds(M + off, L)]
    b2 = x2v[pl.ds(2 * M + off, L)]
    s2v[pl.ds(off, L)] = (b0 * b0 + b1 * b1) + b2 * b2
    x2v[pl.ds(off, L)] = _bf16r(b0)
    x2v[pl.ds(M + off, L)] = _bf16r(b1)
    x2v[pl.ds(2 * M + off, L)] = _bf16r(b2)
    cminv[pl.ds(off, L)] = inf16
    cidxv[pl.ds(off, L)] = zero16

  @plsc.parallel_loop(0, (CHUNK * 3 + L) // L, unroll=4)
  def _prep1(jc):
    off = jc * L
    x1r[pl.ds(off, L)] = _bf16r(x1v[pl.ds(off, L)])

  def _row(r, carry):
    accm, acci, g = carry
    i = g * L + r
    va = x1v[pl.ds(3 * i, L)]
    A0 = jnp.full((L,), va[0], jnp.float32)
    A1 = jnp.full((L,), va[1], jnp.float32)
    A2 = jnp.full((L,), va[2], jnp.float32)
    s1r = (A0 * A0 + A1 * A1) + A2 * A2
    vb = x1r[pl.ds(3 * i, L)]
    a0 = jnp.full((L,), vb[0], jnp.float32)
    a1 = jnp.full((L,), vb[1], jnp.float32)
    a2 = jnp.full((L,), vb[2], jnp.float32)
    iv = jnp.full((L,), row0 + i, jnp.int32)

    @plsc.parallel_loop(0, M // L, carry=(inf16, zero16), unroll=4)
    def _col(jc, carry):
      rmin, ridx = carry
      off = jc * L
      b0 = x2v[pl.ds(off, L)]
      b1 = x2v[pl.ds(M + off, L)]
      b2 = x2v[pl.ds(2 * M + off, L)]
      s2c = s2v[pl.ds(off, L)]
      cross = (a0 * b0 + a1 * b1) + a2 * b2
      d = (s1r + s2c) - 2.0 * cross
      jv = lanes + off
      mr = d < rmin
      rmin = jnp.where(mr, d, rmin)
      ridx = jnp.where(mr, jv, ridx)
      cmin = cminv[pl.ds(off, L)]
      cidx = cidxv[pl.ds(off, L)]
      mc = d < cmin
      cminv[pl.ds(off, L)] = jnp.where(mc, d, cmin)
      cidxv[pl.ds(off, L)] = jnp.where(mc, iv, cidx)
      return rmin, ridx

    rmin, ridx = _col
    rs = jnp.min(rmin)
    ri = jnp.min(jnp.where(rmin == rs, ridx, jnp.int32(M)))
    lm = lanes == r
    accm = jnp.where(lm, rs, accm)
    acci = jnp.where(lm, ri, acci)
    return accm, acci, g

  def _rowgrp(g, _):
    accm, acci, _g = lax.fori_loop(0, L, _row, (inf16, zero16, g))
    rminv[pl.ds(g * L, L)] = accm
    ridxv[pl.ds(g * L, L)] = acci
    return 0
  lax.fori_loop(0, CHUNK // L, _rowgrp, 0)

  pltpu.sync_copy(rminv, d1_hbm.at[b, pl.ds(row0, CHUNK)])
  pltpu.sync_copy(ridxv, i1_hbm.at[b, pl.ds(row0, CHUNK)])

  pltpu.sync_copy(cminv, shmin.at[pl.ds(s * M, M)])
  pltpu.sync_copy(cidxv, shidx.at[pl.ds(s * M, M)])
  plsc.subcore_barrier()

  @pl.when(chunk == 0)
  def _merge():
    pltpu.sync_copy(shmin.at[pl.ds(s * M, WPB * M)], mmin)
    pltpu.sync_copy(shidx.at[pl.ds(s * M, WPB * M)], midx)

    @plsc.parallel_loop(0, M // L, unroll=4)
    def _mrg(jc):
      off = jc * L
      m = mmin[pl.ds(off, L)]
      ix = midx[pl.ds(off, L)]
      for k in range(1, WPB):
        mk = mmin[pl.ds(k * M + off, L)]
        ik = midx[pl.ds(k * M + off, L)]
        lt = mk < m
        m = jnp.where(lt, mk, m)
        ix = jnp.where(lt, ik, ix)
      cminv[pl.ds(off, L)] = m
      cidxv[pl.ds(off, L)] = ix
    pltpu.sync_copy(cminv, d2_hbm.at[b])
    pltpu.sync_copy(cidxv, i2_hbm.at[b])


@jax.jit
def kernel(input1, input2):
  x1f = input1.reshape(B, N * 3)
  x2t = jnp.swapaxes(input2, 1, 2).reshape(B, 3 * M)
  d1, d2, i1, i2 = _chamfer_sc(x1f, x2t)
  return d1, d2, i1, i2


# speedup vs baseline: 6.7194x; 2.2786x over previous
import functools

import jax
import jax.numpy as jnp
from jax import lax
from jax.experimental import pallas as pl
from jax.experimental.pallas import tpu as pltpu
from jax.experimental.pallas import tpu_sc as plsc

NC = 2
NS = 16
L = 16
B = 8
N = 2048
M = 2048

SCB = 2
TCB = B - SCB
TC_R = 512

_mesh = plsc.VectorSubcoreMesh(core_axis_name="c", subcore_axis_name="s",
                               num_cores=NC, num_subcores=NS)


def _make_sc_kernel(nb):
  wpb = NC * NS // nb
  chunk = N // wpb

  @functools.partial(
      pl.kernel,
      out_type=(
          jax.ShapeDtypeStruct((nb, N), jnp.float32),
          jax.ShapeDtypeStruct((nb, M), jnp.float32),
          jax.ShapeDtypeStruct((nb, N), jnp.int32),
          jax.ShapeDtypeStruct((nb, M), jnp.int32),
      ),
      mesh=_mesh,
      compiler_params=pltpu.CompilerParams(needs_layout_passes=False),
      scratch_types=dict(
          x1v=pltpu.VMEM((chunk * 3 + L,), jnp.float32),
          x1r=pltpu.VMEM((chunk * 3 + L,), jnp.float32),
          x2v=pltpu.VMEM((3 * M,), jnp.float32),
          s2v=pltpu.VMEM((M,), jnp.float32),
          rminv=pltpu.VMEM((chunk,), jnp.float32),
          ridxv=pltpu.VMEM((chunk,), jnp.int32),
          cminv=pltpu.VMEM((M,), jnp.float32),
          cidxv=pltpu.VMEM((M,), jnp.int32),
          mmin=pltpu.VMEM((wpb * M,), jnp.float32),
          midx=pltpu.VMEM((wpb * M,), jnp.int32),
          shmin=pltpu.VMEM_SHARED((NS * M,), jnp.float32),
          shidx=pltpu.VMEM_SHARED((NS * M,), jnp.int32),
      ),
  )
  def _chamfer_sc(x1_hbm, x2_hbm, d1_hbm, d2_hbm, i1_hbm, i2_hbm,
                  x1v, x1r, x2v, s2v, rminv, ridxv, cminv, cidxv,
                  mmin, midx, shmin, shidx):
    c = lax.axis_index("c")
    s = lax.axis_index("s")
    wid = c * NS + s
    b = wid // wpb
    ch = wid % wpb
    row0 = ch * chunk

    pltpu.sync_copy(x1_hbm.at[b, pl.ds(row0 * 3, chunk * 3)],
                    x1v.at[pl.ds(0, chunk * 3)])
    pltpu.sync_copy(x2_hbm.at[b], x2v)

    lanes = lax.iota(jnp.int32, L)
    inf16 = jnp.full((L,), jnp.inf, jnp.float32)
    zero16 = jnp.zeros((L,), jnp.int32)

    def _bf16r(v):
      u = plsc.bitcast(v, jnp.uint32)
      u = (u + jnp.uint32(0x7FFF) + ((u >> jnp.uint32(16)) & jnp.uint32(1)))
      u = u & jnp.uint32(0xFFFF0000)
      return plsc.bitcast(u, jnp.float32)

    @plsc.parallel_loop(0, M // L, unroll=4)
    def _prep(jc):
      off = jc * L
      b0 = x2v[pl.ds(off, L)]
      b1 = x2v[pl.ds(M + off, L)]
      b2 = x2v[pl.ds(2 * M + off, L)]
      s2v[pl.ds(off, L)] = (b0 * b0 + b1 * b1) + b2 * b2
      x2v[pl.ds(off, L)] = _bf16r(b0)
      x2v[pl.ds(M + off, L)] = _bf16r(b1)
      x2v[pl.ds(2 * M + off, L)] = _bf16r(b2)
      cminv[pl.ds(off, L)] = inf16
      cidxv[pl.ds(off, L)] = zero16

    @plsc.parallel_loop(0, (chunk * 3 + L) // L, unroll=4)
    def _prep1(jc):
      off = jc * L
      x1r[pl.ds(off, L)] = _bf16r(x1v[pl.ds(off, L)])

    def _row(r, carry):
      accm, acci, g = carry
      i = g * L + r
      va = x1v[pl.ds(3 * i, L)]
      A0 = jnp.full((L,), va[0], jnp.float32)
      A1 = jnp.full((L,), va[1], jnp.float32)
      A2 = jnp.full((L,), va[2], jnp.float32)
      s1r = (A0 * A0 + A1 * A1) + A2 * A2
      vb = x1r[pl.ds(3 * i, L)]
      a0 = jnp.full((L,), vb[0], jnp.float32)
      a1 = jnp.full((L,), vb[1], jnp.float32)
      a2 = jnp.full((L,), vb[2], jnp.float32)
      iv = jnp.full((L,), row0 + i, jnp.int32)

      @plsc.parallel_loop(0, M // L, carry=(inf16, zero16), unroll=4)
      def _col(jc, carry2):
        rmin, ridx = carry2
        off = jc * L
        b0 = x2v[pl.ds(off, L)]
        b1 = x2v[pl.ds(M + off, L)]
        b2 = x2v[pl.ds(2 * M + off, L)]
        s2c = s2v[pl.ds(off, L)]
        cross = (a0 * b0 + a1 * b1) + a2 * b2
        d = (s1r + s2c) - 2.0 * cross
        jv = lanes + off
        mr = d < rmin
        rmin = jnp.where(mr, d, rmin)
        ridx = jnp.where(mr, jv, ridx)
        cmin = cminv[pl.ds(off, L)]
        cidx = cidxv[pl.ds(off, L)]
        mc = d < cmin
        cminv[pl.ds(off, L)] = jnp.where(mc, d, cmin)
        cidxv[pl.ds(off, L)] = jnp.where(mc, iv, cidx)
        return rmin, ridx

      rmin, ridx = _col
      rs = jnp.min(rmin)
      ri = jnp.min(jnp.where(rmin == rs, ridx, jnp.int32(M)))
      lm = lanes == r
      accm = jnp.where(lm, rs, accm)
      acci = jnp.where(lm, ri, acci)
      return accm, acci, g

    def _rowgrp(g, _):
      accm, acci, _g = lax.fori_loop(0, L, _row, (inf16, zero16, g))
      rminv[pl.ds(g * L, L)] = accm
      ridxv[pl.ds(g * L, L)] = acci
      return 0
    lax.fori_loop(0, chunk // L, _rowgrp, 0)

    pltpu.sync_copy(rminv, d1_hbm.at[b, pl.ds(row0, chunk)])
    pltpu.sync_copy(ridxv, i1_hbm.at[b, pl.ds(row0, chunk)])

    pltpu.sync_copy(cminv, shmin.at[pl.ds(s * M, M)])
    pltpu.sync_copy(cidxv, shidx.at[pl.ds(s * M, M)])
    plsc.subcore_barrier()

    @pl.when(ch == 0)
    def _merge():
      pltpu.sync_copy(shmin.at[pl.ds(s * M, wpb * M)], mmin)
      pltpu.sync_copy(shidx.at[pl.ds(s * M, wpb * M)], midx)

      @plsc.parallel_loop(0, M // L, unroll=4)
      def _mrg(jc):
        off = jc * L
        m = mmin[pl.ds(off, L)]
        ix = midx[pl.ds(off, L)]
        for k in range(1, wpb):
          mk = mmin[pl.ds(k * M + off, L)]
          ik = midx[pl.ds(k * M + off, L)]
          lt = mk < m
          m = jnp.where(lt, mk, m)
          ix = jnp.where(lt, ik, ix)
        cminv[pl.ds(off, L)] = m
        cidxv[pl.ds(off, L)] = ix
      pltpu.sync_copy(cminv, d2_hbm.at[b])
      pltpu.sync_copy(cidxv, i2_hbm.at[b])

  return _chamfer_sc


_chamfer_sc = _make_sc_kernel(SCB)


def _tc_rtne(u):
  v = lax.bitcast_convert_type(u, jnp.uint32)
  v = v + jnp.uint32(0x7FFF) + ((v >> jnp.uint32(16)) & jnp.uint32(1))
  v = v & jnp.uint32(0xFFFF0000)
  return lax.bitcast_convert_type(v, jnp.float32)


def _tc_body(x1_ref, x2_ref, d1_ref, d2_ref, i1_ref, i2_ref, cminp, cidxp):
  t = pl.program_id(1)
  nt = pl.num_programs(1)
  a = x1_ref[0]
  A0, A1, A2 = a[:, 0:1], a[:, 1:2], a[:, 2:3]
  s1 = (A0 * A0 + A1 * A1) + A2 * A2
  bb = x2_ref[0]
  B0, B1, B2 = bb[0:1, :], bb[1:2, :], bb[2:3, :]
  s2 = (B0 * B0 + B1 * B1) + B2 * B2
  a0, a1, a2 = _tc_rtne(A0), _tc_rtne(A1), _tc_rtne(A2)
  b0, b1, b2 = _tc_rtne(B0), _tc_rtne(B1), _tc_rtne(B2)
  cross = (a0 * b0 + a1 * b1) + a2 * b2
  d = (s1 + s2) - 2.0 * cross

  jiota = lax.broadcasted_iota(jnp.int32, (TC_R, M), 1)
  rmin = jnp.min(d, axis=1, keepdims=True)
  ridx = jnp.min(jnp.where(d == rmin, jiota, M), axis=1, keepdims=True)
  d1_ref[0] = rmin
  i1_ref[0] = ridx

  riota = lax.broadcasted_iota(jnp.int32, (TC_R, M), 0) + t * TC_R
  tcmin = jnp.min(d, axis=0, keepdims=True)
  tcidx = jnp.min(jnp.where(d == tcmin, riota, N), axis=0, keepdims=True)

  @pl.when(t == 0)
  def _():
    cminp[...] = jnp.full((1, M), jnp.inf, jnp.float32)
    cidxp[...] = jnp.zeros((1, M), jnp.int32)

  upd = tcmin < cminp[...]
  cminp[...] = jnp.where(upd, tcmin, cminp[...])
  cidxp[...] = jnp.where(upd, tcidx, cidxp[...])

  @pl.when(t == nt - 1)
  def _():
    d2_ref[0] = cminp[...]
    i2_ref[0] = cidxp[...]


def _chamfer_tc(x1, x2t):
  nb = x1.shape[0]
  nt = N // TC_R
  out = pl.pallas_call(
      _tc_body,
      grid=(nb, nt),
      in_specs=[
          pl.BlockSpec((1, TC_R, 3), lambda b, t: (b, t, 0)),
          pl.BlockSpec((1, 3, M), lambda b, t: (b, 0, 0)),
      ],
      out_specs=[
          pl.BlockSpec((1, TC_R, 1), lambda b, t: (b, t, 0)),
          pl.BlockSpec((1, 1, M), lambda b, t: (b, 0, 0)),
          pl.BlockSpec((1, TC_R, 1), lambda b, t: (b, t, 0)),
          pl.BlockSpec((1, 1, M), lambda b, t: (b, 0, 0)),
      ],
      out_shape=[
          jax.ShapeDtypeStruct((nb, N, 1), jnp.float32),
          jax.ShapeDtypeStruct((nb, 1, M), jnp.float32),
          jax.ShapeDtypeStruct((nb, N, 1), jnp.int32),
          jax.ShapeDtypeStruct((nb, 1, M), jnp.int32),
      ],
      scratch_shapes=[
          pltpu.VMEM((1, M), jnp.float32),
          pltpu.VMEM((1, M), jnp.int32),
      ],
  )(x1, x2t)
  d1, d2, i1, i2 = out
  return (d1.reshape(nb, N), d2.reshape(nb, M),
          i1.reshape(nb, N), i2.reshape(nb, M))


@jax.jit
def kernel(input1, input2):
  x2t = jnp.swapaxes(input2, 1, 2)
  x1f_sc = input1[:SCB].reshape(SCB, N * 3)
  x2t_sc = x2t[:SCB].reshape(SCB, 3 * M)
  sd1, sd2, si1, si2 = _chamfer_sc(x1f_sc, x2t_sc)
  td1, td2, ti1, ti2 = _chamfer_tc(input1[SCB:], x2t[SCB:])
  d1 = jnp.concatenate([sd1, td1], axis=0)
  d2 = jnp.concatenate([sd2, td2], axis=0)
  i1 = jnp.concatenate([si1, ti1], axis=0)
  i2 = jnp.concatenate([si2, ti2], axis=0)
  return d1, d2, i1, i2


# speedup vs baseline: 7.7474x; 1.1530x over previous
import functools

import jax
import jax.numpy as jnp
from jax import lax
from jax.experimental import pallas as pl
from jax.experimental.pallas import tpu as pltpu
from jax.experimental.pallas import tpu_sc as plsc

NC = 2
NS = 16
L = 16
B = 8
N = 2048
M = 2048

SCB = 2
TCB = B - SCB
TC_R = 512

_mesh = plsc.VectorSubcoreMesh(core_axis_name="c", subcore_axis_name="s",
                               num_cores=NC, num_subcores=NS)


def _make_sc_kernel(nb):
  wpb = NC * NS // nb
  chunk = N // wpb

  @functools.partial(
      pl.kernel,
      out_type=(
          jax.ShapeDtypeStruct((nb, N), jnp.float32),
          jax.ShapeDtypeStruct((nb, M), jnp.float32),
          jax.ShapeDtypeStruct((nb, N), jnp.int32),
          jax.ShapeDtypeStruct((nb, M), jnp.int32),
      ),
      mesh=_mesh,
      compiler_params=pltpu.CompilerParams(needs_layout_passes=False),
      scratch_types=dict(
          x1v=pltpu.VMEM((chunk * 3 + L,), jnp.float32),
          x1r=pltpu.VMEM((chunk * 3 + L,), jnp.float32),
          x2v=pltpu.VMEM((3 * M,), jnp.float32),
          s2v=pltpu.VMEM((M,), jnp.float32),
          rminv=pltpu.VMEM((chunk,), jnp.float32),
          ridxv=pltpu.VMEM((chunk,), jnp.int32),
          cminv=pltpu.VMEM((M,), jnp.float32),
          cidxv=pltpu.VMEM((M,), jnp.int32),
          mmin=pltpu.VMEM((wpb * M,), jnp.float32),
          midx=pltpu.VMEM((wpb * M,), jnp.int32),
          shmin=pltpu.VMEM_SHARED((NS * M,), jnp.float32),
          shidx=pltpu.VMEM_SHARED((NS * M,), jnp.int32),
      ),
  )
  def _chamfer_sc(x1_hbm, x2_hbm, d1_hbm, d2_hbm, i1_hbm, i2_hbm,
                  x1v, x1r, x2v, s2v, rminv, ridxv, cminv, cidxv,
                  mmin, midx, shmin, shidx):
    c = lax.axis_index("c")
    s = lax.axis_index("s")
    wid = c * NS + s
    b = wid // wpb
    ch = wid % wpb
    row0 = ch * chunk

    pltpu.sync_copy(x1_hbm.at[b, pl.ds(row0 * 3, chunk * 3)],
                    x1v.at[pl.ds(0, chunk * 3)])
    pltpu.sync_copy(x2_hbm.at[b], x2v)

    lanes = lax.iota(jnp.int32, L)
    inf16 = jnp.full((L,), jnp.inf, jnp.float32)
    zero16 = jnp.zeros((L,), jnp.int32)

    def _bf16r(v):
      u = plsc.bitcast(v, jnp.uint32)
      u = (u + jnp.uint32(0x7FFF) + ((u >> jnp.uint32(16)) & jnp.uint32(1)))
      u = u & jnp.uint32(0xFFFF0000)
      return plsc.bitcast(u, jnp.float32)

    @plsc.parallel_loop(0, M // L, unroll=4)
    def _prep(jc):
      off = jc * L
      b0 = x2v[pl.ds(off, L)]
      b1 = x2v[pl.ds(M + off, L)]
      b2 = x2v[pl.ds(2 * M + off, L)]
      s2v[pl.ds(off, L)] = (b0 * b0 + b1 * b1) + b2 * b2
      x2v[pl.ds(off, L)] = _bf16r(b0)
      x2v[pl.ds(M + off, L)] = _bf16r(b1)
      x2v[pl.ds(2 * M + off, L)] = _bf16r(b2)
      cminv[pl.ds(off, L)] = inf16
      cidxv[pl.ds(off, L)] = zero16

    @plsc.parallel_loop(0, (chunk * 3 + L) // L, unroll=4)
    def _prep1(jc):
      off = jc * L
      x1r[pl.ds(off, L)] = _bf16r(x1v[pl.ds(off, L)])

    def _row(r, carry):
      accm, acci, g = carry
      i = g * L + r
      va = x1v[pl.ds(3 * i, L)]
      A0 = jnp.full((L,), va[0], jnp.float32)
      A1 = jnp.full((L,), va[1], jnp.float32)
      A2 = jnp.full((L,), va[2], jnp.float32)
      s1r = (A0 * A0 + A1 * A1) + A2 * A2
      vb = x1r[pl.ds(3 * i, L)]
      a0 = jnp.full((L,), vb[0], jnp.float32)
      a1 = jnp.full((L,), vb[1], jnp.float32)
      a2 = jnp.full((L,), vb[2], jnp.float32)
      iv = jnp.full((L,), row0 + i, jnp.int32)

      @plsc.parallel_loop(0, M // L, carry=(inf16, zero16), unroll=4)
      def _col(jc, carry2):
        rmin, ridx = carry2
        off = jc * L
        b0 = x2v[pl.ds(off, L)]
        b1 = x2v[pl.ds(M + off, L)]
        b2 = x2v[pl.ds(2 * M + off, L)]
        s2c = s2v[pl.ds(off, L)]
        cross = (a0 * b0 + a1 * b1) + a2 * b2
        d = (s1r + s2c) - 2.0 * cross
        jv = lanes + off
        mr = d < rmin
        rmin = jnp.where(mr, d, rmin)
        ridx = jnp.where(mr, jv, ridx)
        cmin = cminv[pl.ds(off, L)]
        cidx = cidxv[pl.ds(off, L)]
        mc = d < cmin
        cminv[pl.ds(off, L)] = jnp.where(mc, d, cmin)
        cidxv[pl.ds(off, L)] = jnp.where(mc, iv, cidx)
        return rmin, ridx

      rmin, ridx = _col
      rs = jnp.min(rmin)
      ri = jnp.min(jnp.where(rmin == rs, ridx, jnp.int32(M)))
      lm = lanes == r
      accm = jnp.where(lm, rs, accm)
      acci = jnp.where(lm, ri, acci)
      return accm, acci, g

    def _rowgrp(g, _):
      accm, acci, _g = lax.fori_loop(0, L, _row, (inf16, zero16, g))
      rminv[pl.ds(g * L, L)] = accm
      ridxv[pl.ds(g * L, L)] = acci
      return 0
    lax.fori_loop(0, chunk // L, _rowgrp, 0)

    pltpu.sync_copy(rminv, d1_hbm.at[b, pl.ds(row0, chunk)])
    pltpu.sync_copy(ridxv, i1_hbm.at[b, pl.ds(row0, chunk)])

    pltpu.sync_copy(cminv, shmin.at[pl.ds(s * M, M)])
    pltpu.sync_copy(cidxv, shidx.at[pl.ds(s * M, M)])
    plsc.subcore_barrier()

    @pl.when(ch == 0)
    def _merge():
      pltpu.sync_copy(shmin.at[pl.ds(s * M, wpb * M)], mmin)
      pltpu.sync_copy(shidx.at[pl.ds(s * M, wpb * M)], midx)

      @plsc.parallel_loop(0, M // L, unroll=4)
      def _mrg(jc):
        off = jc * L
        m = mmin[pl.ds(off, L)]
        ix = midx[pl.ds(off, L)]
        for k in range(1, wpb):
          mk = mmin[pl.ds(k * M + off, L)]
          ik = midx[pl.ds(k * M + off, L)]
          lt = mk < m
          m = jnp.where(lt, mk, m)
          ix = jnp.where(lt, ik, ix)
        cminv[pl.ds(off, L)] = m
        cidxv[pl.ds(off, L)] = ix
      pltpu.sync_copy(cminv, d2_hbm.at[b])
      pltpu.sync_copy(cidxv, i2_hbm.at[b])

  return _chamfer_sc


_chamfer_sc = _make_sc_kernel(SCB)


def _tc_body(x1_ref, x2_ref, d1_ref, d2_ref, i1_ref, i2_ref, cminp, cidxp):
  t = pl.program_id(1)
  nt = pl.num_programs(1)
  a = x1_ref[0]
  A0, A1, A2 = a[:, 0:1], a[:, 1:2], a[:, 2:3]
  s1 = (A0 * A0 + A1 * A1) + A2 * A2
  bb = x2_ref[0]
  B0, B1, B2 = bb[0:1, :], bb[1:2, :], bb[2:3, :]
  s2 = (B0 * B0 + B1 * B1) + B2 * B2
  cross = lax.dot_general(a, bb, (((1,), (0,)), ((), ())),
                          preferred_element_type=jnp.float32)
  d = (s1 + s2) - 2.0 * cross

  jiota = lax.broadcasted_iota(jnp.int32, (TC_R, M), 1)
  rmin = jnp.min(d, axis=1, keepdims=True)
  ridx = jnp.min(jnp.where(d == rmin, jiota, M), axis=1, keepdims=True)
  d1_ref[0] = rmin
  i1_ref[0] = ridx

  riota = lax.broadcasted_iota(jnp.int32, (TC_R, M), 0) + t * TC_R
  tcmin = jnp.min(d, axis=0, keepdims=True)
  tcidx = jnp.min(jnp.where(d == tcmin, riota, N), axis=0, keepdims=True)

  @pl.when(t == 0)
  def _():
    cminp[...] = jnp.full((1, M), jnp.inf, jnp.float32)
    cidxp[...] = jnp.zeros((1, M), jnp.int32)

  upd = tcmin < cminp[...]
  cminp[...] = jnp.where(upd, tcmin, cminp[...])
  cidxp[...] = jnp.where(upd, tcidx, cidxp[...])

  @pl.when(t == nt - 1)
  def _():
    d2_ref[0] = cminp[...]
    i2_ref[0] = cidxp[...]


def _chamfer_tc(x1, x2t):
  nb = x1.shape[0]
  nt = N // TC_R
  out = pl.pallas_call(
      _tc_body,
      grid=(nb, nt),
      in_specs=[
          pl.BlockSpec((1, TC_R, 3), lambda b, t: (b, t, 0)),
          pl.BlockSpec((1, 3, M), lambda b, t: (b, 0, 0)),
      ],
      out_specs=[
          pl.BlockSpec((1, TC_R, 1), lambda b, t: (b, t, 0)),
          pl.BlockSpec((1, 1, M), lambda b, t: (b, 0, 0)),
          pl.BlockSpec((1, TC_R, 1), lambda b, t: (b, t, 0)),
          pl.BlockSpec((1, 1, M), lambda b, t: (b, 0, 0)),
      ],
      out_shape=[
          jax.ShapeDtypeStruct((nb, N, 1), jnp.float32),
          jax.ShapeDtypeStruct((nb, 1, M), jnp.float32),
          jax.ShapeDtypeStruct((nb, N, 1), jnp.int32),
          jax.ShapeDtypeStruct((nb, 1, M), jnp.int32),
      ],
      scratch_shapes=[
          pltpu.VMEM((1, M), jnp.float32),
          pltpu.VMEM((1, M), jnp.int32),
      ],
  )(x1, x2t)
  d1, d2, i1, i2 = out
  return (d1.reshape(nb, N), d2.reshape(nb, M),
          i1.reshape(nb, N), i2.reshape(nb, M))


@jax.jit
def kernel(input1, input2):
  x2t = jnp.swapaxes(input2, 1, 2)
  x1f_sc = input1[:SCB].reshape(SCB, N * 3)
  x2t_sc = x2t[:SCB].reshape(SCB, 3 * M)
  sd1, sd2, si1, si2 = _chamfer_sc(x1f_sc, x2t_sc)
  td1, td2, ti1, ti2 = _chamfer_tc(input1[SCB:], x2t[SCB:])
  d1 = jnp.concatenate([sd1, td1], axis=0)
  d2 = jnp.concatenate([sd2, td2], axis=0)
  i1 = jnp.concatenate([si1, ti1], axis=0)
  i2 = jnp.concatenate([si2, ti2], axis=0)
  return d1, d2, i1, i2


# speedup vs baseline: 7.8794x; 1.0170x over previous
import functools

import jax
import jax.numpy as jnp
from jax import lax
from jax.experimental import pallas as pl
from jax.experimental.pallas import tpu as pltpu
from jax.experimental.pallas import tpu_sc as plsc

NC = 2
NS = 16
L = 16
B = 8
N = 2048
M = 2048

SCB = 2
TCB = B - SCB
TC_R = 1024

_mesh = plsc.VectorSubcoreMesh(core_axis_name="c", subcore_axis_name="s",
                               num_cores=NC, num_subcores=NS)


def _make_sc_kernel(nb):
  wpb = NC * NS // nb
  chunk = N // wpb

  @functools.partial(
      pl.kernel,
      out_type=(
          jax.ShapeDtypeStruct((nb, N), jnp.float32),
          jax.ShapeDtypeStruct((nb, M), jnp.float32),
          jax.ShapeDtypeStruct((nb, N), jnp.int32),
          jax.ShapeDtypeStruct((nb, M), jnp.int32),
      ),
      mesh=_mesh,
      compiler_params=pltpu.CompilerParams(needs_layout_passes=False),
      scratch_types=dict(
          x1v=pltpu.VMEM((chunk * 3 + L,), jnp.float32),
          x1r=pltpu.VMEM((chunk * 3 + L,), jnp.float32),
          x2v=pltpu.VMEM((3 * M,), jnp.float32),
          s2v=pltpu.VMEM((M,), jnp.float32),
          rminv=pltpu.VMEM((chunk,), jnp.float32),
          ridxv=pltpu.VMEM((chunk,), jnp.int32),
          cminv=pltpu.VMEM((M,), jnp.float32),
          cidxv=pltpu.VMEM((M,), jnp.int32),
          mmin=pltpu.VMEM((wpb * M,), jnp.float32),
          midx=pltpu.VMEM((wpb * M,), jnp.int32),
          shmin=pltpu.VMEM_SHARED((NS * M,), jnp.float32),
          shidx=pltpu.VMEM_SHARED((NS * M,), jnp.int32),
      ),
  )
  def _chamfer_sc(x1_hbm, x2_hbm, d1_hbm, d2_hbm, i1_hbm, i2_hbm,
                  x1v, x1r, x2v, s2v, rminv, ridxv, cminv, cidxv,
                  mmin, midx, shmin, shidx):
    c = lax.axis_index("c")
    s = lax.axis_index("s")
    wid = c * NS + s
    b = wid // wpb
    ch = wid % wpb
    row0 = ch * chunk

    pltpu.sync_copy(x1_hbm.at[b, pl.ds(row0 * 3, chunk * 3)],
                    x1v.at[pl.ds(0, chunk * 3)])
    pltpu.sync_copy(x2_hbm.at[b], x2v)

    lanes = lax.iota(jnp.int32, L)
    inf16 = jnp.full((L,), jnp.inf, jnp.float32)
    zero16 = jnp.zeros((L,), jnp.int32)

    def _bf16r(v):
      u = plsc.bitcast(v, jnp.uint32)
      u = (u + jnp.uint32(0x7FFF) + ((u >> jnp.uint32(16)) & jnp.uint32(1)))
      u = u & jnp.uint32(0xFFFF0000)
      return plsc.bitcast(u, jnp.float32)

    @plsc.parallel_loop(0, M // L, unroll=4)
    def _prep(jc):
      off = jc * L
      b0 = x2v[pl.ds(off, L)]
      b1 = x2v[pl.ds(M + off, L)]
      b2 = x2v[pl.ds(2 * M + off, L)]
      s2v[pl.ds(off, L)] = (b0 * b0 + b1 * b1) + b2 * b2
      x2v[pl.ds(off, L)] = _bf16r(b0)
      x2v[pl.ds(M + off, L)] = _bf16r(b1)
      x2v[pl.ds(2 * M + off, L)] = _bf16r(b2)
      cminv[pl.ds(off, L)] = inf16
      cidxv[pl.ds(off, L)] = zero16

    @plsc.parallel_loop(0, (chunk * 3 + L) // L, unroll=4)
    def _prep1(jc):
      off = jc * L
      x1r[pl.ds(off, L)] = _bf16r(x1v[pl.ds(off, L)])

    def _row(r, carry):
      accm, acci, g = carry
      i = g * L + r
      va = x1v[pl.ds(3 * i, L)]
      A0 = jnp.full((L,), va[0], jnp.float32)
      A1 = jnp.full((L,), va[1], jnp.float32)
      A2 = jnp.full((L,), va[2], jnp.float32)
      s1r = (A0 * A0 + A1 * A1) + A2 * A2
      vb = x1r[pl.ds(3 * i, L)]
      a0 = jnp.full((L,), vb[0], jnp.float32)
      a1 = jnp.full((L,), vb[1], jnp.float32)
      a2 = jnp.full((L,), vb[2], jnp.float32)
      iv = jnp.full((L,), row0 + i, jnp.int32)

      @plsc.parallel_loop(0, M // L, carry=(inf16, zero16), unroll=4)
      def _col(jc, carry2):
        rmin, ridx = carry2
        off = jc * L
        b0 = x2v[pl.ds(off, L)]
        b1 = x2v[pl.ds(M + off, L)]
        b2 = x2v[pl.ds(2 * M + off, L)]
        s2c = s2v[pl.ds(off, L)]
        cross = (a0 * b0 + a1 * b1) + a2 * b2
        d = (s1r + s2c) - 2.0 * cross
        jv = lanes + off
        mr = d < rmin
        rmin = jnp.where(mr, d, rmin)
        ridx = jnp.where(mr, jv, ridx)
        cmin = cminv[pl.ds(off, L)]
        cidx = cidxv[pl.ds(off, L)]
        mc = d < cmin
        cminv[pl.ds(off, L)] = jnp.where(mc, d, cmin)
        cidxv[pl.ds(off, L)] = jnp.where(mc, iv, cidx)
        return rmin, ridx

      rmin, ridx = _col
      rs = jnp.min(rmin)
      ri = jnp.min(jnp.where(rmin == rs, ridx, jnp.int32(M)))
      lm = lanes == r
      accm = jnp.where(lm, rs, accm)
      acci = jnp.where(lm, ri, acci)
      return accm, acci, g

    def _rowgrp(g, _):
      accm, acci, _g = lax.fori_loop(0, L, _row, (inf16, zero16, g))
      rminv[pl.ds(g * L, L)] = accm
      ridxv[pl.ds(g * L, L)] = acci
      return 0
    lax.fori_loop(0, chunk // L, _rowgrp, 0)

    pltpu.sync_copy(rminv, d1_hbm.at[b, pl.ds(row0, chunk)])
    pltpu.sync_copy(ridxv, i1_hbm.at[b, pl.ds(row0, chunk)])

    pltpu.sync_copy(cminv, shmin.at[pl.ds(s * M, M)])
    pltpu.sync_copy(cidxv, shidx.at[pl.ds(s * M, M)])
    plsc.subcore_barrier()

    @pl.when(ch == 0)
    def _merge():
      pltpu.sync_copy(shmin.at[pl.ds(s * M, wpb * M)], mmin)
      pltpu.sync_copy(shidx.at[pl.ds(s * M, wpb * M)], midx)

      @plsc.parallel_loop(0, M // L, unroll=4)
      def _mrg(jc):
        off = jc * L
        m = mmin[pl.ds(off, L)]
        ix = midx[pl.ds(off, L)]
        for k in range(1, wpb):
          mk = mmin[pl.ds(k * M + off, L)]
          ik = midx[pl.ds(k * M + off, L)]
          lt = mk < m
          m = jnp.where(lt, mk, m)
          ix = jnp.where(lt, ik, ix)
        cminv[pl.ds(off, L)] = m
        cidxv[pl.ds(off, L)] = ix
      pltpu.sync_copy(cminv, d2_hbm.at[b])
      pltpu.sync_copy(cidxv, i2_hbm.at[b])

  return _chamfer_sc


_chamfer_sc = _make_sc_kernel(SCB)


def _tc_body(x1_ref, x2_ref, d1_ref, d2_ref, i1_ref, i2_ref, cminp, cidxp):
  t = pl.program_id(1)
  nt = pl.num_programs(1)
  a = x1_ref[0]
  A0, A1, A2 = a[:, 0:1], a[:, 1:2], a[:, 2:3]
  s1 = (A0 * A0 + A1 * A1) + A2 * A2
  bb = x2_ref[0]
  B0, B1, B2 = bb[0:1, :], bb[1:2, :], bb[2:3, :]
  s2 = (B0 * B0 + B1 * B1) + B2 * B2
  cross = lax.dot_general(a, bb, (((1,), (0,)), ((), ())),
                          preferred_element_type=jnp.float32)
  d = (s1 + s2) - 2.0 * cross

  jiota = lax.broadcasted_iota(jnp.int32, (TC_R, M), 1)
  rmin = jnp.min(d, axis=1, keepdims=True)
  ridx = jnp.min(jnp.where(d == rmin, jiota, M), axis=1, keepdims=True)
  d1_ref[0] = rmin
  i1_ref[0] = ridx

  riota = lax.broadcasted_iota(jnp.int32, (TC_R, M), 0) + t * TC_R
  tcmin = jnp.min(d, axis=0, keepdims=True)
  tcidx = jnp.min(jnp.where(d == tcmin, riota, N), axis=0, keepdims=True)

  @pl.when(t == 0)
  def _():
    cminp[...] = jnp.full((1, M), jnp.inf, jnp.float32)
    cidxp[...] = jnp.zeros((1, M), jnp.int32)

  upd = tcmin < cminp[...]
  cminp[...] = jnp.where(upd, tcmin, cminp[...])
  cidxp[...] = jnp.where(upd, tcidx, cidxp[...])

  @pl.when(t == nt - 1)
  def _():
    d2_ref[0] = cminp[...]
    i2_ref[0] = cidxp[...]


def _chamfer_tc(x1, x2t):
  nb = x1.shape[0]
  nt = N // TC_R
  out = pl.pallas_call(
      _tc_body,
      grid=(nb, nt),
      in_specs=[
          pl.BlockSpec((1, TC_R, 3), lambda b, t: (b, t, 0)),
          pl.BlockSpec((1, 3, M), lambda b, t: (b, 0, 0)),
      ],
      out_specs=[
          pl.BlockSpec((1, TC_R, 1), lambda b, t: (b, t, 0)),
          pl.BlockSpec((1, 1, M), lambda b, t: (b, 0, 0)),
          pl.BlockSpec((1, TC_R, 1), lambda b, t: (b, t, 0)),
          pl.BlockSpec((1, 1, M), lambda b, t: (b, 0, 0)),
      ],
      out_shape=[
          jax.ShapeDtypeStruct((nb, N, 1), jnp.float32),
          jax.ShapeDtypeStruct((nb, 1, M), jnp.float32),
          jax.ShapeDtypeStruct((nb, N, 1), jnp.int32),
          jax.ShapeDtypeStruct((nb, 1, M), jnp.int32),
      ],
      scratch_shapes=[
          pltpu.VMEM((1, M), jnp.float32),
          pltpu.VMEM((1, M), jnp.int32),
      ],
  )(x1, x2t)
  d1, d2, i1, i2 = out
  return (d1.reshape(nb, N), d2.reshape(nb, M),
          i1.reshape(nb, N), i2.reshape(nb, M))


@jax.jit
def kernel(input1, input2):
  x2t = jnp.swapaxes(input2, 1, 2)
  x1f_sc = input1[:SCB].reshape(SCB, N * 3)
  x2t_sc = x2t[:SCB].reshape(SCB, 3 * M)
  sd1, sd2, si1, si2 = _chamfer_sc(x1f_sc, x2t_sc)
  td1, td2, ti1, ti2 = _chamfer_tc(input1[SCB:], x2t[SCB:])
  d1 = jnp.concatenate([sd1, td1], axis=0)
  d2 = jnp.concatenate([sd2, td2], axis=0)
  i1 = jnp.concatenate([si1, ti1], axis=0)
  i2 = jnp.concatenate([si2, ti2], axis=0)
  return d1, d2, i1, i2


# speedup vs baseline: 8.2163x; 1.0428x over previous
import functools

import jax
import jax.numpy as jnp
from jax import lax
from jax.experimental import pallas as pl
from jax.experimental.pallas import tpu as pltpu
from jax.experimental.pallas import tpu_sc as plsc

NC = 2
NS = 16
L = 16
B = 8
N = 2048
M = 2048

SCB = 2
TCB = B - SCB
TC_R = 1024

_mesh = plsc.VectorSubcoreMesh(core_axis_name="c", subcore_axis_name="s",
                               num_cores=NC, num_subcores=NS)


def _make_sc_kernel(nb):
  wpb = NC * NS // nb
  chunk = N // wpb

  @functools.partial(
      pl.kernel,
      out_type=(
          jax.ShapeDtypeStruct((nb, N), jnp.float32),
          jax.ShapeDtypeStruct((nb, M), jnp.float32),
          jax.ShapeDtypeStruct((nb, N), jnp.int32),
          jax.ShapeDtypeStruct((nb, M), jnp.int32),
      ),
      mesh=_mesh,
      compiler_params=pltpu.CompilerParams(needs_layout_passes=False),
      scratch_types=dict(
          x1v=pltpu.VMEM((chunk * 3 + L,), jnp.float32),
          x1r=pltpu.VMEM((chunk * 3 + L,), jnp.float32),
          x2v=pltpu.VMEM((3 * M,), jnp.float32),
          s2v=pltpu.VMEM((M,), jnp.float32),
          rminv=pltpu.VMEM((chunk,), jnp.float32),
          ridxv=pltpu.VMEM((chunk,), jnp.int32),
          cminv=pltpu.VMEM((M,), jnp.float32),
          cidxv=pltpu.VMEM((M,), jnp.int32),
          mmin=pltpu.VMEM((wpb * M,), jnp.float32),
          midx=pltpu.VMEM((wpb * M,), jnp.int32),
          shmin=pltpu.VMEM_SHARED((NS * M,), jnp.float32),
          shidx=pltpu.VMEM_SHARED((NS * M,), jnp.int32),
      ),
  )
  def _chamfer_sc(x1_hbm, x2_hbm, d1_hbm, d2_hbm, i1_hbm, i2_hbm,
                  x1v, x1r, x2v, s2v, rminv, ridxv, cminv, cidxv,
                  mmin, midx, shmin, shidx):
    c = lax.axis_index("c")
    s = lax.axis_index("s")
    wid = c * NS + s
    b = wid // wpb
    ch = wid % wpb
    row0 = ch * chunk

    pltpu.sync_copy(x1_hbm.at[b, pl.ds(row0 * 3, chunk * 3)],
                    x1v.at[pl.ds(0, chunk * 3)])
    pltpu.sync_copy(x2_hbm.at[b], x2v)

    lanes = lax.iota(jnp.int32, L)
    inf16 = jnp.full((L,), jnp.inf, jnp.float32)
    zero16 = jnp.zeros((L,), jnp.int32)

    def _bf16r(v):
      u = plsc.bitcast(v, jnp.uint32)
      u = (u + jnp.uint32(0x7FFF) + ((u >> jnp.uint32(16)) & jnp.uint32(1)))
      u = u & jnp.uint32(0xFFFF0000)
      return plsc.bitcast(u, jnp.float32)

    @plsc.parallel_loop(0, M // L, unroll=4)
    def _prep(jc):
      off = jc * L
      b0 = x2v[pl.ds(off, L)]
      b1 = x2v[pl.ds(M + off, L)]
      b2 = x2v[pl.ds(2 * M + off, L)]
      s2v[pl.ds(off, L)] = (b0 * b0 + b1 * b1) + b2 * b2
      x2v[pl.ds(off, L)] = _bf16r(b0)
      x2v[pl.ds(M + off, L)] = _bf16r(b1)
      x2v[pl.ds(2 * M + off, L)] = _bf16r(b2)
      cminv[pl.ds(off, L)] = inf16
      cidxv[pl.ds(off, L)] = zero16

    @plsc.parallel_loop(0, (chunk * 3 + L) // L, unroll=4)
    def _prep1(jc):
      off = jc * L
      x1r[pl.ds(off, L)] = _bf16r(x1v[pl.ds(off, L)])

    def _row(r, carry):
      accm, acci, g = carry
      i = g * L + r
      va = x1v[pl.ds(3 * i, L)]
      A0 = jnp.full((L,), va[0], jnp.float32)
      A1 = jnp.full((L,), va[1], jnp.float32)
      A2 = jnp.full((L,), va[2], jnp.float32)
      s1r = (A0 * A0 + A1 * A1) + A2 * A2
      vb = x1r[pl.ds(3 * i, L)]
      a0 = jnp.full((L,), vb[0], jnp.float32)
      a1 = jnp.full((L,), vb[1], jnp.float32)
      a2 = jnp.full((L,), vb[2], jnp.float32)
      iv = jnp.full((L,), row0 + i, jnp.int32)

      @plsc.parallel_loop(0, M // L, carry=(inf16, zero16), unroll=4)
      def _col(jc, carry2):
        rmin, ridx = carry2
        off = jc * L
        b0 = x2v[pl.ds(off, L)]
        b1 = x2v[pl.ds(M + off, L)]
        b2 = x2v[pl.ds(2 * M + off, L)]
        s2c = s2v[pl.ds(off, L)]
        cross = (a0 * b0 + a1 * b1) + a2 * b2
        d = (s1r + s2c) - 2.0 * cross
        jv = lanes + off
        mr = d < rmin
        rmin = jnp.where(mr, d, rmin)
        ridx = jnp.where(mr, jv, ridx)
        cmin = cminv[pl.ds(off, L)]
        cidx = cidxv[pl.ds(off, L)]
        mc = d < cmin
        cminv[pl.ds(off, L)] = jnp.where(mc, d, cmin)
        cidxv[pl.ds(off, L)] = jnp.where(mc, iv, cidx)
        return rmin, ridx

      rmin, ridx = _col
      rs = jnp.min(rmin)
      ri = jnp.min(jnp.where(rmin == rs, ridx, jnp.int32(M)))
      lm = lanes == r
      accm = jnp.where(lm, rs, accm)
      acci = jnp.where(lm, ri, acci)
      return accm, acci, g

    def _rowgrp(g, _):
      accm, acci, _g = lax.fori_loop(0, L, _row, (inf16, zero16, g))
      rminv[pl.ds(g * L, L)] = accm
      ridxv[pl.ds(g * L, L)] = acci
      return 0
    lax.fori_loop(0, chunk // L, _rowgrp, 0)

    pltpu.sync_copy(rminv, d1_hbm.at[b, pl.ds(row0, chunk)])
    pltpu.sync_copy(ridxv, i1_hbm.at[b, pl.ds(row0, chunk)])

    pltpu.sync_copy(cminv, shmin.at[pl.ds(s * M, M)])
    pltpu.sync_copy(cidxv, shidx.at[pl.ds(s * M, M)])
    plsc.subcore_barrier()

    @pl.when(ch == 0)
    def _merge():
      pltpu.sync_copy(shmin.at[pl.ds(s * M, wpb * M)], mmin)
      pltpu.sync_copy(shidx.at[pl.ds(s * M, wpb * M)], midx)

      @plsc.parallel_loop(0, M // L, unroll=4)
      def _mrg(jc):
        off = jc * L
        m = mmin[pl.ds(off, L)]
        ix = midx[pl.ds(off, L)]
        for k in range(1, wpb):
          mk = mmin[pl.ds(k * M + off, L)]
          ik = midx[pl.ds(k * M + off, L)]
          lt = mk < m
          m = jnp.where(lt, mk, m)
          ix = jnp.where(lt, ik, ix)
        cminv[pl.ds(off, L)] = m
        cidxv[pl.ds(off, L)] = ix
      pltpu.sync_copy(cminv, d2_hbm.at[b])
      pltpu.sync_copy(cidxv, i2_hbm.at[b])

  return _chamfer_sc


_chamfer_sc = _make_sc_kernel(SCB)


def _tc_body(x1_ref, x2_ref, d1_ref, d2_ref, i1_ref, i2_ref, cminp, cidxp):
  t = pl.program_id(1)
  nt = pl.num_programs(1)
  a = x1_ref[0]
  A0, A1, A2 = a[:, 0:1], a[:, 1:2], a[:, 2:3]
  s1 = (A0 * A0 + A1 * A1) + A2 * A2
  bb = x2_ref[0]
  B0, B1, B2 = bb[0:1, :], bb[1:2, :], bb[2:3, :]
  s2 = (B0 * B0 + B1 * B1) + B2 * B2
  ncross2 = lax.dot_general(-2.0 * a, bb, (((1,), (0,)), ((), ())),
                            preferred_element_type=jnp.float32)
  d = (s1 + s2) + ncross2

  jiota = lax.broadcasted_iota(jnp.int32, (TC_R, M), 1).astype(jnp.float32)
  rmin = jnp.min(d, axis=1, keepdims=True)
  ridx = jnp.min(jnp.where(d == rmin, jiota, float(M)), axis=1, keepdims=True)
  d1_ref[0] = rmin
  i1_ref[0] = ridx.astype(jnp.int32)

  riota = (lax.broadcasted_iota(jnp.int32, (TC_R, M), 0).astype(jnp.float32)
           + (t * TC_R).astype(jnp.float32))
  tcmin = jnp.min(d, axis=0, keepdims=True)
  tcidx = jnp.min(jnp.where(d == tcmin, riota, float(N)), axis=0, keepdims=True)

  @pl.when(t == 0)
  def _():
    cminp[...] = jnp.full((1, M), jnp.inf, jnp.float32)
    cidxp[...] = jnp.zeros((1, M), jnp.float32)

  upd = tcmin < cminp[...]
  cminp[...] = jnp.where(upd, tcmin, cminp[...])
  cidxp[...] = jnp.where(upd, tcidx, cidxp[...])

  @pl.when(t == nt - 1)
  def _():
    d2_ref[0] = cminp[...]
    i2_ref[0] = cidxp[...].astype(jnp.int32)


def _chamfer_tc(x1, x2t):
  nb = x1.shape[0]
  nt = N // TC_R
  out = pl.pallas_call(
      _tc_body,
      grid=(nb, nt),
      in_specs=[
          pl.BlockSpec((1, TC_R, 3), lambda b, t: (b, t, 0)),
          pl.BlockSpec((1, 3, M), lambda b, t: (b, 0, 0)),
      ],
      out_specs=[
          pl.BlockSpec((1, TC_R, 1), lambda b, t: (b, t, 0)),
          pl.BlockSpec((1, 1, M), lambda b, t: (b, 0, 0)),
          pl.BlockSpec((1, TC_R, 1), lambda b, t: (b, t, 0)),
          pl.BlockSpec((1, 1, M), lambda b, t: (b, 0, 0)),
      ],
      out_shape=[
          jax.ShapeDtypeStruct((nb, N, 1), jnp.float32),
          jax.ShapeDtypeStruct((nb, 1, M), jnp.float32),
          jax.ShapeDtypeStruct((nb, N, 1), jnp.int32),
          jax.ShapeDtypeStruct((nb, 1, M), jnp.int32),
      ],
      scratch_shapes=[
          pltpu.VMEM((1, M), jnp.float32),
          pltpu.VMEM((1, M), jnp.float32),
      ],
  )(x1, x2t)
  d1, d2, i1, i2 = out
  return (d1.reshape(nb, N), d2.reshape(nb, M),
          i1.reshape(nb, N), i2.reshape(nb, M))


@jax.jit
def kernel(input1, input2):
  x2t = jnp.swapaxes(input2, 1, 2)
  x1f_sc = input1[:SCB].reshape(SCB, N * 3)
  x2t_sc = x2t[:SCB].reshape(SCB, 3 * M)
  sd1, sd2, si1, si2 = _chamfer_sc(x1f_sc, x2t_sc)
  td1, td2, ti1, ti2 = _chamfer_tc(input1[SCB:], x2t[SCB:])
  d1 = jnp.concatenate([sd1, td1], axis=0)
  d2 = jnp.concatenate([sd2, td2], axis=0)
  i1 = jnp.concatenate([si1, ti1], axis=0)
  i2 = jnp.concatenate([si2, ti2], axis=0)
  return d1, d2, i1, i2


# speedup vs baseline: 8.5991x; 1.0466x over previous
import functools

import jax
import jax.numpy as jnp
from jax import lax
from jax.experimental import pallas as pl
from jax.experimental.pallas import tpu as pltpu
from jax.experimental.pallas import tpu_sc as plsc

NC = 2
NS = 16
L = 16
B = 8
N = 2048
M = 2048

SCB = 2
TCB = B - SCB
TC_R = 1024

_mesh = plsc.VectorSubcoreMesh(core_axis_name="c", subcore_axis_name="s",
                               num_cores=NC, num_subcores=NS)


def _make_sc_kernel(nb):
  wpb = NC * NS // nb
  chunk = N // wpb

  @functools.partial(
      pl.kernel,
      out_type=(
          jax.ShapeDtypeStruct((nb, N), jnp.float32),
          jax.ShapeDtypeStruct((nb, M), jnp.float32),
          jax.ShapeDtypeStruct((nb, N), jnp.int32),
          jax.ShapeDtypeStruct((nb, M), jnp.int32),
      ),
      mesh=_mesh,
      compiler_params=pltpu.CompilerParams(needs_layout_passes=False),
      scratch_types=dict(
          x1v=pltpu.VMEM((chunk * 3 + L,), jnp.float32),
          x1r=pltpu.VMEM((chunk * 3 + L,), jnp.float32),
          x2v=pltpu.VMEM((3 * M,), jnp.float32),
          s2v=pltpu.VMEM((M,), jnp.float32),
          rminv=pltpu.VMEM((chunk,), jnp.float32),
          ridxv=pltpu.VMEM((chunk,), jnp.int32),
          cminv=pltpu.VMEM((M,), jnp.float32),
          cidxv=pltpu.VMEM((M,), jnp.int32),
          mmin=pltpu.VMEM((wpb * M,), jnp.float32),
          midx=pltpu.VMEM((wpb * M,), jnp.int32),
          shmin=pltpu.VMEM_SHARED((NS * M,), jnp.float32),
          shidx=pltpu.VMEM_SHARED((NS * M,), jnp.int32),
      ),
  )
  def _chamfer_sc(x1_hbm, x2_hbm, d1_hbm, d2_hbm, i1_hbm, i2_hbm,
                  x1v, x1r, x2v, s2v, rminv, ridxv, cminv, cidxv,
                  mmin, midx, shmin, shidx):
    c = lax.axis_index("c")
    s = lax.axis_index("s")
    wid = c * NS + s
    b = wid // wpb
    ch = wid % wpb
    row0 = ch * chunk

    pltpu.sync_copy(x1_hbm.at[b, pl.ds(row0 * 3, chunk * 3)],
                    x1v.at[pl.ds(0, chunk * 3)])
    pltpu.sync_copy(x2_hbm.at[b], x2v)

    lanes = lax.iota(jnp.int32, L)
    inf16 = jnp.full((L,), jnp.inf, jnp.float32)
    zero16 = jnp.zeros((L,), jnp.int32)

    def _bf16r(v):
      u = plsc.bitcast(v, jnp.uint32)
      u = (u + jnp.uint32(0x7FFF) + ((u >> jnp.uint32(16)) & jnp.uint32(1)))
      u = u & jnp.uint32(0xFFFF0000)
      return plsc.bitcast(u, jnp.float32)

    @plsc.parallel_loop(0, M // L, unroll=4)
    def _prep(jc):
      off = jc * L
      b0 = x2v[pl.ds(off, L)]
      b1 = x2v[pl.ds(M + off, L)]
      b2 = x2v[pl.ds(2 * M + off, L)]
      s2v[pl.ds(off, L)] = (b0 * b0 + b1 * b1) + b2 * b2
      x2v[pl.ds(off, L)] = _bf16r(b0)
      x2v[pl.ds(M + off, L)] = _bf16r(b1)
      x2v[pl.ds(2 * M + off, L)] = _bf16r(b2)
      cminv[pl.ds(off, L)] = inf16
      cidxv[pl.ds(off, L)] = zero16

    @plsc.parallel_loop(0, (chunk * 3 + L) // L, unroll=4)
    def _prep1(jc):
      off = jc * L
      x1r[pl.ds(off, L)] = _bf16r(x1v[pl.ds(off, L)])

    RB = 4

    def _quad(q, carry):
      accm, acci, g = carry
      i0 = g * L + q * RB
      s1s, a0s, a1s, a2s, ivs = [], [], [], [], []
      for r in range(RB):
        va = x1v[pl.ds(3 * (i0 + r), L)]
        A0 = jnp.full((L,), va[0], jnp.float32)
        A1 = jnp.full((L,), va[1], jnp.float32)
        A2 = jnp.full((L,), va[2], jnp.float32)
        s1s.append((A0 * A0 + A1 * A1) + A2 * A2)
        vb = x1r[pl.ds(3 * (i0 + r), L)]
        a0s.append(jnp.full((L,), vb[0], jnp.float32))
        a1s.append(jnp.full((L,), vb[1], jnp.float32))
        a2s.append(jnp.full((L,), vb[2], jnp.float32))
        ivs.append(jnp.full((L,), row0 + i0 + r, jnp.int32))

      init = tuple([inf16] * RB + [zero16] * RB)

      @plsc.parallel_loop(0, M // L, carry=init, unroll=2)
      def _col(jc, carry2):
        rmins = list(carry2[:RB])
        ridxs = list(carry2[RB:])
        off = jc * L
        b0 = x2v[pl.ds(off, L)]
        b1 = x2v[pl.ds(M + off, L)]
        b2 = x2v[pl.ds(2 * M + off, L)]
        s2c = s2v[pl.ds(off, L)]
        jv = lanes + off
        cmin = cminv[pl.ds(off, L)]
        cidx = cidxv[pl.ds(off, L)]
        for r in range(RB):
          cross = (a0s[r] * b0 + a1s[r] * b1) + a2s[r] * b2
          d = (s1s[r] + s2c) - 2.0 * cross
          mr = d < rmins[r]
          rmins[r] = jnp.where(mr, d, rmins[r])
          ridxs[r] = jnp.where(mr, jv, ridxs[r])
          mc = d < cmin
          cmin = jnp.where(mc, d, cmin)
          cidx = jnp.where(mc, ivs[r], cidx)
        cminv[pl.ds(off, L)] = cmin
        cidxv[pl.ds(off, L)] = cidx
        return tuple(rmins + ridxs)

      res = _col
      for r in range(RB):
        rmin, ridx = res[r], res[RB + r]
        rs = jnp.min(rmin)
        ri = jnp.min(jnp.where(rmin == rs, ridx, jnp.int32(M)))
        lm = lanes == q * RB + r
        accm = jnp.where(lm, rs, accm)
        acci = jnp.where(lm, ri, acci)
      return accm, acci, g

    def _rowgrp(g, _):
      accm, acci, _g = lax.fori_loop(0, L // RB, _quad, (inf16, zero16, g))
      rminv[pl.ds(g * L, L)] = accm
      ridxv[pl.ds(g * L, L)] = acci
      return 0
    lax.fori_loop(0, chunk // L, _rowgrp, 0)

    pltpu.sync_copy(rminv, d1_hbm.at[b, pl.ds(row0, chunk)])
    pltpu.sync_copy(ridxv, i1_hbm.at[b, pl.ds(row0, chunk)])

    pltpu.sync_copy(cminv, shmin.at[pl.ds(s * M, M)])
    pltpu.sync_copy(cidxv, shidx.at[pl.ds(s * M, M)])
    plsc.subcore_barrier()

    @pl.when(ch == 0)
    def _merge():
      pltpu.sync_copy(shmin.at[pl.ds(s * M, wpb * M)], mmin)
      pltpu.sync_copy(shidx.at[pl.ds(s * M, wpb * M)], midx)

      @plsc.parallel_loop(0, M // L, unroll=4)
      def _mrg(jc):
        off = jc * L
        m = mmin[pl.ds(off, L)]
        ix = midx[pl.ds(off, L)]
        for k in range(1, wpb):
          mk = mmin[pl.ds(k * M + off, L)]
          ik = midx[pl.ds(k * M + off, L)]
          lt = mk < m
          m = jnp.where(lt, mk, m)
          ix = jnp.where(lt, ik, ix)
        cminv[pl.ds(off, L)] = m
        cidxv[pl.ds(off, L)] = ix
      pltpu.sync_copy(cminv, d2_hbm.at[b])
      pltpu.sync_copy(cidxv, i2_hbm.at[b])

  return _chamfer_sc


_chamfer_sc = _make_sc_kernel(SCB)


def _tc_body(x1_ref, x2_ref, d1_ref, d2_ref, i1_ref, i2_ref, cminp, cidxp):
  t = pl.program_id(1)
  nt = pl.num_programs(1)
  a = x1_ref[0]
  A0, A1, A2 = a[:, 0:1], a[:, 1:2], a[:, 2:3]
  s1 = (A0 * A0 + A1 * A1) + A2 * A2
  bb = x2_ref[0]
  B0, B1, B2 = bb[0:1, :], bb[1:2, :], bb[2:3, :]
  s2 = (B0 * B0 + B1 * B1) + B2 * B2
  ncross2 = lax.dot_general(-2.0 * a, bb, (((1,), (0,)), ((), ())),
                            preferred_element_type=jnp.float32)
  d = (s1 + s2) + ncross2

  jiota = lax.broadcasted_iota(jnp.int32, (TC_R, M), 1).astype(jnp.float32)
  rmin = jnp.min(d, axis=1, keepdims=True)
  ridx = jnp.min(jnp.where(d == rmin, jiota, float(M)), axis=1, keepdims=True)
  d1_ref[0] = rmin
  i1_ref[0] = ridx.astype(jnp.int32)

  riota = (lax.broadcasted_iota(jnp.int32, (TC_R, M), 0).astype(jnp.float32)
           + (t * TC_R).astype(jnp.float32))
  tcmin = jnp.min(d, axis=0, keepdims=True)
  tcidx = jnp.min(jnp.where(d == tcmin, riota, float(N)), axis=0, keepdims=True)

  @pl.when(t == 0)
  def _():
    cminp[...] = jnp.full((1, M), jnp.inf, jnp.float32)
    cidxp[...] = jnp.zeros((1, M), jnp.float32)

  upd = tcmin < cminp[...]
  cminp[...] = jnp.where(upd, tcmin, cminp[...])
  cidxp[...] = jnp.where(upd, tcidx, cidxp[...])

  @pl.when(t == nt - 1)
  def _():
    d2_ref[0] = cminp[...]
    i2_ref[0] = cidxp[...].astype(jnp.int32)


def _chamfer_tc(x1, x2t):
  nb = x1.shape[0]
  nt = N // TC_R
  out = pl.pallas_call(
      _tc_body,
      grid=(nb, nt),
      in_specs=[
          pl.BlockSpec((1, TC_R, 3), lambda b, t: (b, t, 0)),
          pl.BlockSpec((1, 3, M), lambda b, t: (b, 0, 0)),
      ],
      out_specs=[
          pl.BlockSpec((1, TC_R, 1), lambda b, t: (b, t, 0)),
          pl.BlockSpec((1, 1, M), lambda b, t: (b, 0, 0)),
          pl.BlockSpec((1, TC_R, 1), lambda b, t: (b, t, 0)),
          pl.BlockSpec((1, 1, M), lambda b, t: (b, 0, 0)),
      ],
      out_shape=[
          jax.ShapeDtypeStruct((nb, N, 1), jnp.float32),
          jax.ShapeDtypeStruct((nb, 1, M), jnp.float32),
          jax.ShapeDtypeStruct((nb, N, 1), jnp.int32),
          jax.ShapeDtypeStruct((nb, 1, M), jnp.int32),
      ],
      scratch_shapes=[
          pltpu.VMEM((1, M), jnp.float32),
          pltpu.VMEM((1, M), jnp.float32),
      ],
  )(x1, x2t)
  d1, d2, i1, i2 = out
  return (d1.reshape(nb, N), d2.reshape(nb, M),
          i1.reshape(nb, N), i2.reshape(nb, M))


@jax.jit
def kernel(input1, input2):
  x2t = jnp.swapaxes(input2, 1, 2)
  x1f_sc = input1[:SCB].reshape(SCB, N * 3)
  x2t_sc = x2t[:SCB].reshape(SCB, 3 * M)
  sd1, sd2, si1, si2 = _chamfer_sc(x1f_sc, x2t_sc)
  td1, td2, ti1, ti2 = _chamfer_tc(input1[SCB:], x2t[SCB:])
  d1 = jnp.concatenate([sd1, td1], axis=0)
  d2 = jnp.concatenate([sd2, td2], axis=0)
  i1 = jnp.concatenate([si1, ti1], axis=0)
  i2 = jnp.concatenate([si2, ti2], axis=0)
  return d1, d2, i1, i2


# speedup vs baseline: 8.7982x; 1.0232x over previous
import functools

import jax
import jax.numpy as jnp
from jax import lax
from jax.experimental import pallas as pl
from jax.experimental.pallas import tpu as pltpu
from jax.experimental.pallas import tpu_sc as plsc

NC = 2
NS = 16
L = 16
B = 8
N = 2048
M = 2048

SCB = 2
TCB = B - SCB
TC_R = 1024

_mesh = plsc.VectorSubcoreMesh(core_axis_name="c", subcore_axis_name="s",
                               num_cores=NC, num_subcores=NS)


def _make_sc_kernel(nb):
  wpb = NC * NS // nb
  chunk = N // wpb

  @functools.partial(
      pl.kernel,
      out_type=(
          jax.ShapeDtypeStruct((nb, N), jnp.float32),
          jax.ShapeDtypeStruct((nb, M), jnp.float32),
          jax.ShapeDtypeStruct((nb, N), jnp.int32),
          jax.ShapeDtypeStruct((nb, M), jnp.int32),
      ),
      mesh=_mesh,
      compiler_params=pltpu.CompilerParams(needs_layout_passes=False),
      scratch_types=dict(
          x1v=pltpu.VMEM((chunk * 3 + L,), jnp.float32),
          x1r=pltpu.VMEM((chunk * 3 + L,), jnp.float32),
          x2v=pltpu.VMEM((3 * M,), jnp.float32),
          s2v=pltpu.VMEM((M,), jnp.float32),
          rminv=pltpu.VMEM((chunk,), jnp.float32),
          ridxv=pltpu.VMEM((chunk,), jnp.int32),
          cminv=pltpu.VMEM((M,), jnp.float32),
          cidxv=pltpu.VMEM((M,), jnp.int32),
          mmin=pltpu.VMEM((wpb * M,), jnp.float32),
          midx=pltpu.VMEM((wpb * M,), jnp.int32),
          shmin=pltpu.VMEM_SHARED((NS * M,), jnp.float32),
          shidx=pltpu.VMEM_SHARED((NS * M,), jnp.int32),
      ),
  )
  def _chamfer_sc(x1_hbm, x2_hbm, d1_hbm, d2_hbm, i1_hbm, i2_hbm,
                  x1v, x1r, x2v, s2v, rminv, ridxv, cminv, cidxv,
                  mmin, midx, shmin, shidx):
    c = lax.axis_index("c")
    s = lax.axis_index("s")
    wid = c * NS + s
    b = wid // wpb
    ch = wid % wpb
    row0 = ch * chunk

    pltpu.sync_copy(x1_hbm.at[b, pl.ds(row0 * 3, chunk * 3)],
                    x1v.at[pl.ds(0, chunk * 3)])
    pltpu.sync_copy(x2_hbm.at[b], x2v)

    lanes = lax.iota(jnp.int32, L)
    inf16 = jnp.full((L,), jnp.inf, jnp.float32)
    zero16 = jnp.zeros((L,), jnp.int32)

    def _bf16r(v):
      u = plsc.bitcast(v, jnp.uint32)
      u = (u + jnp.uint32(0x7FFF) + ((u >> jnp.uint32(16)) & jnp.uint32(1)))
      u = u & jnp.uint32(0xFFFF0000)
      return plsc.bitcast(u, jnp.float32)

    @plsc.parallel_loop(0, M // L, unroll=4)
    def _prep(jc):
      off = jc * L
      b0 = x2v[pl.ds(off, L)]
      b1 = x2v[pl.ds(M + off, L)]
      b2 = x2v[pl.ds(2 * M + off, L)]
      s2v[pl.ds(off, L)] = (b0 * b0 + b1 * b1) + b2 * b2
      x2v[pl.ds(off, L)] = _bf16r(b0)
      x2v[pl.ds(M + off, L)] = _bf16r(b1)
      x2v[pl.ds(2 * M + off, L)] = _bf16r(b2)
      cminv[pl.ds(off, L)] = inf16
      cidxv[pl.ds(off, L)] = zero16

    @plsc.parallel_loop(0, (chunk * 3 + L) // L, unroll=4)
    def _prep1(jc):
      off = jc * L
      x1r[pl.ds(off, L)] = _bf16r(x1v[pl.ds(off, L)])

    RB = 4

    def _quad(q, carry):
      accm, acci, g = carry
      i0 = g * L + q * RB
      s1s, a0s, a1s, a2s, ivs = [], [], [], [], []
      for r in range(RB):
        va = x1v[pl.ds(3 * (i0 + r), L)]
        A0 = jnp.full((L,), va[0], jnp.float32)
        A1 = jnp.full((L,), va[1], jnp.float32)
        A2 = jnp.full((L,), va[2], jnp.float32)
        s1s.append((A0 * A0 + A1 * A1) + A2 * A2)
        vb = x1r[pl.ds(3 * (i0 + r), L)]
        a0s.append(jnp.full((L,), vb[0], jnp.float32))
        a1s.append(jnp.full((L,), vb[1], jnp.float32))
        a2s.append(jnp.full((L,), vb[2], jnp.float32))
        ivs.append(jnp.full((L,), row0 + i0 + r, jnp.int32))

      init = tuple([inf16] * RB + [zero16] * RB)

      @plsc.parallel_loop(0, M // L, carry=init, unroll=2)
      def _col(jc, carry2):
        rmins = list(carry2[:RB])
        ridxs = list(carry2[RB:])
        off = jc * L
        b0 = x2v[pl.ds(off, L)]
        b1 = x2v[pl.ds(M + off, L)]
        b2 = x2v[pl.ds(2 * M + off, L)]
        s2c = s2v[pl.ds(off, L)]
        jv = lanes + off
        cmin = cminv[pl.ds(off, L)]
        cidx = cidxv[pl.ds(off, L)]
        for r in range(RB):
          cross = (a0s[r] * b0 + a1s[r] * b1) + a2s[r] * b2
          d = (s1s[r] + s2c) - 2.0 * cross
          mr = d < rmins[r]
          rmins[r] = jnp.where(mr, d, rmins[r])
          ridxs[r] = jnp.where(mr, jv, ridxs[r])
          mc = d < cmin
          cmin = jnp.where(mc, d, cmin)
          cidx = jnp.where(mc, ivs[r], cidx)
        cminv[pl.ds(off, L)] = cmin
        cidxv[pl.ds(off, L)] = cidx
        return tuple(rmins + ridxs)

      res = _col
      for r in range(RB):
        rmin, ridx = res[r], res[RB + r]
        rs = jnp.min(rmin)
        ri = jnp.min(jnp.where(rmin == rs, ridx, jnp.int32(M)))
        lm = lanes == q * RB + r
        accm = jnp.where(lm, rs, accm)
        acci = jnp.where(lm, ri, acci)
      return accm, acci, g

    def _rowgrp(g, _):
      accm, acci, _g = lax.fori_loop(0, L // RB, _quad, (inf16, zero16, g))
      rminv[pl.ds(g * L, L)] = accm
      ridxv[pl.ds(g * L, L)] = acci
      return 0
    lax.fori_loop(0, chunk // L, _rowgrp, 0)

    pltpu.sync_copy(rminv, d1_hbm.at[b, pl.ds(row0, chunk)])
    pltpu.sync_copy(ridxv, i1_hbm.at[b, pl.ds(row0, chunk)])

    pltpu.sync_copy(cminv, shmin.at[pl.ds(s * M, M)])
    pltpu.sync_copy(cidxv, shidx.at[pl.ds(s * M, M)])
    plsc.subcore_barrier()

    @pl.when(ch == 0)
    def _merge():
      pltpu.sync_copy(shmin.at[pl.ds(s * M, wpb * M)], mmin)
      pltpu.sync_copy(shidx.at[pl.ds(s * M, wpb * M)], midx)

      @plsc.parallel_loop(0, M // L, unroll=4)
      def _mrg(jc):
        off = jc * L
        m = mmin[pl.ds(off, L)]
        ix = midx[pl.ds(off, L)]
        for k in range(1, wpb):
          mk = mmin[pl.ds(k * M + off, L)]
          ik = midx[pl.ds(k * M + off, L)]
          lt = mk < m
          m = jnp.where(lt, mk, m)
          ix = jnp.where(lt, ik, ix)
        cminv[pl.ds(off, L)] = m
        cidxv[pl.ds(off, L)] = ix
      pltpu.sync_copy(cminv, d2_hbm.at[b])
      pltpu.sync_copy(cidxv, i2_hbm.at[b])

  return _chamfer_sc


_chamfer_sc = _make_sc_kernel(SCB)


def _tc_body(x1_ref, x2_ref, d1_ref, d2_ref, i1_ref, i2_ref, cminp, cidxp):
  t = pl.program_id(1)
  nt = pl.num_programs(1)
  a = x1_ref[0]
  A0, A1, A2 = a[:, 0:1], a[:, 1:2], a[:, 2:3]
  s1 = (A0 * A0 + A1 * A1) + A2 * A2
  bb = x2_ref[0]
  B0, B1, B2 = bb[0:1, :], bb[1:2, :], bb[2:3, :]
  s2 = (B0 * B0 + B1 * B1) + B2 * B2
  ncross2 = lax.dot_general(-2.0 * a, bb, (((1,), (0,)), ((), ())),
                            preferred_element_type=jnp.float32)
  d = (s1 + s2) + ncross2

  jiota = lax.broadcasted_iota(jnp.int32, (TC_R, M), 1).astype(jnp.float32)
  rmin = jnp.min(d, axis=1, keepdims=True)
  ridx = jnp.min(jnp.where(d == rmin, jiota, float(M)), axis=1, keepdims=True)
  d1_ref[0] = rmin
  i1_ref[0] = ridx.astype(jnp.int32)

  riota = (lax.broadcasted_iota(jnp.int32, (TC_R, M), 0).astype(jnp.float32)
           + (t * TC_R).astype(jnp.float32))
  tcmin = jnp.min(d, axis=0, keepdims=True)
  tcidx = jnp.min(jnp.where(d == tcmin, riota, float(N)), axis=0, keepdims=True)

  @pl.when(t == 0)
  def _():
    cminp[...] = jnp.full((1, M), jnp.inf, jnp.float32)
    cidxp[...] = jnp.zeros((1, M), jnp.float32)

  upd = tcmin < cminp[...]
  cminp[...] = jnp.where(upd, tcmin, cminp[...])
  cidxp[...] = jnp.where(upd, tcidx, cidxp[...])

  @pl.when(t == nt - 1)
  def _():
    d2_ref[0] = cminp[...]
    i2_ref[0] = cidxp[...].astype(jnp.int32)


def _chamfer_tc(x1, x2t):
  nb = x1.shape[0]
  nt = N // TC_R
  out = pl.pallas_call(
      _tc_body,
      grid=(nb, nt),
      in_specs=[
          pl.BlockSpec((1, TC_R, 3), lambda b, t: (b, t, 0)),
          pl.BlockSpec((1, 3, M), lambda b, t: (b, 0, 0)),
      ],
      out_specs=[
          pl.BlockSpec((1, TC_R, 1), lambda b, t: (b, t, 0)),
          pl.BlockSpec((1, 1, M), lambda b, t: (b, 0, 0)),
          pl.BlockSpec((1, TC_R, 1), lambda b, t: (b, t, 0)),
          pl.BlockSpec((1, 1, M), lambda b, t: (b, 0, 0)),
      ],
      out_shape=[
          jax.ShapeDtypeStruct((nb, N, 1), jnp.float32),
          jax.ShapeDtypeStruct((nb, 1, M), jnp.float32),
          jax.ShapeDtypeStruct((nb, N, 1), jnp.int32),
          jax.ShapeDtypeStruct((nb, 1, M), jnp.int32),
      ],
      scratch_shapes=[
          pltpu.VMEM((1, M), jnp.float32),
          pltpu.VMEM((1, M), jnp.float32),
      ],
  )(x1, x2t)
  d1, d2, i1, i2 = out
  return (d1.reshape(nb, N), d2.reshape(nb, M),
          i1.reshape(nb, N), i2.reshape(nb, M))


@jax.jit
def kernel(input1, input2):
  x2t = jnp.swapaxes(input2, 1, 2)
  return _chamfer_tc(input1, x2t)


@jax.jit
def _kernel_hybrid(input1, input2):
  x2t = jnp.swapaxes(input2, 1, 2)
  x1f_sc = input1[:SCB].reshape(SCB, N * 3)
  x2t_sc = x2t[:SCB].reshape(SCB, 3 * M)
  sd1, sd2, si1, si2 = _chamfer_sc(x1f_sc, x2t_sc)
  td1, td2, ti1, ti2 = _chamfer_tc(input1[SCB:], x2t[SCB:])
  d1 = jnp.concatenate([sd1, td1], axis=0)
  d2 = jnp.concatenate([sd2, td2], axis=0)
  i1 = jnp.concatenate([si1, ti1], axis=0)
  i2 = jnp.concatenate([si2, ti2], axis=0)
  return d1, d2, i1, i2


# speedup vs baseline: 9.6984x; 1.1023x over previous
import functools

import jax
import jax.numpy as jnp
from jax import lax
from jax.experimental import pallas as pl
from jax.experimental.pallas import tpu as pltpu
from jax.experimental.pallas import tpu_sc as plsc

NC = 2
NS = 16
L = 16
B = 8
N = 2048
M = 2048

SCB = 2
TCB = B - SCB
TC_R = 2048

_mesh = plsc.VectorSubcoreMesh(core_axis_name="c", subcore_axis_name="s",
                               num_cores=NC, num_subcores=NS)


def _make_sc_kernel(nb):
  wpb = NC * NS // nb
  chunk = N // wpb

  @functools.partial(
      pl.kernel,
      out_type=(
          jax.ShapeDtypeStruct((nb, N), jnp.float32),
          jax.ShapeDtypeStruct((nb, M), jnp.float32),
          jax.ShapeDtypeStruct((nb, N), jnp.int32),
          jax.ShapeDtypeStruct((nb, M), jnp.int32),
      ),
      mesh=_mesh,
      compiler_params=pltpu.CompilerParams(needs_layout_passes=False),
      scratch_types=dict(
          x1v=pltpu.VMEM((chunk * 3 + L,), jnp.float32),
          x1r=pltpu.VMEM((chunk * 3 + L,), jnp.float32),
          x2v=pltpu.VMEM((3 * M,), jnp.float32),
          s2v=pltpu.VMEM((M,), jnp.float32),
          rminv=pltpu.VMEM((chunk,), jnp.float32),
          ridxv=pltpu.VMEM((chunk,), jnp.int32),
          cminv=pltpu.VMEM((M,), jnp.float32),
          cidxv=pltpu.VMEM((M,), jnp.int32),
          mmin=pltpu.VMEM((wpb * M,), jnp.float32),
          midx=pltpu.VMEM((wpb * M,), jnp.int32),
          shmin=pltpu.VMEM_SHARED((NS * M,), jnp.float32),
          shidx=pltpu.VMEM_SHARED((NS * M,), jnp.int32),
      ),
  )
  def _chamfer_sc(x1_hbm, x2_hbm, d1_hbm, d2_hbm, i1_hbm, i2_hbm,
                  x1v, x1r, x2v, s2v, rminv, ridxv, cminv, cidxv,
                  mmin, midx, shmin, shidx):
    c = lax.axis_index("c")
    s = lax.axis_index("s")
    wid = c * NS + s
    b = wid // wpb
    ch = wid % wpb
    row0 = ch * chunk

    pltpu.sync_copy(x1_hbm.at[b, pl.ds(row0 * 3, chunk * 3)],
                    x1v.at[pl.ds(0, chunk * 3)])
    pltpu.sync_copy(x2_hbm.at[b], x2v)

    lanes = lax.iota(jnp.int32, L)
    inf16 = jnp.full((L,), jnp.inf, jnp.float32)
    zero16 = jnp.zeros((L,), jnp.int32)

    def _bf16r(v):
      u = plsc.bitcast(v, jnp.uint32)
      u = (u + jnp.uint32(0x7FFF) + ((u >> jnp.uint32(16)) & jnp.uint32(1)))
      u = u & jnp.uint32(0xFFFF0000)
      return plsc.bitcast(u, jnp.float32)

    @plsc.parallel_loop(0, M // L, unroll=4)
    def _prep(jc):
      off = jc * L
      b0 = x2v[pl.ds(off, L)]
      b1 = x2v[pl.ds(M + off, L)]
      b2 = x2v[pl.ds(2 * M + off, L)]
      s2v[pl.ds(off, L)] = (b0 * b0 + b1 * b1) + b2 * b2
      x2v[pl.ds(off, L)] = _bf16r(b0)
      x2v[pl.ds(M + off, L)] = _bf16r(b1)
      x2v[pl.ds(2 * M + off, L)] = _bf16r(b2)
      cminv[pl.ds(off, L)] = inf16
      cidxv[pl.ds(off, L)] = zero16

    @plsc.parallel_loop(0, (chunk * 3 + L) // L, unroll=4)
    def _prep1(jc):
      off = jc * L
      x1r[pl.ds(off, L)] = _bf16r(x1v[pl.ds(off, L)])

    RB = 4

    def _quad(q, carry):
      accm, acci, g = carry
      i0 = g * L + q * RB
      s1s, a0s, a1s, a2s, ivs = [], [], [], [], []
      for r in range(RB):
        va = x1v[pl.ds(3 * (i0 + r), L)]
        A0 = jnp.full((L,), va[0], jnp.float32)
        A1 = jnp.full((L,), va[1], jnp.float32)
        A2 = jnp.full((L,), va[2], jnp.float32)
        s1s.append((A0 * A0 + A1 * A1) + A2 * A2)
        vb = x1r[pl.ds(3 * (i0 + r), L)]
        a0s.append(jnp.full((L,), vb[0], jnp.float32))
        a1s.append(jnp.full((L,), vb[1], jnp.float32))
        a2s.append(jnp.full((L,), vb[2], jnp.float32))
        ivs.append(jnp.full((L,), row0 + i0 + r, jnp.int32))

      init = tuple([inf16] * RB + [zero16] * RB)

      @plsc.parallel_loop(0, M // L, carry=init, unroll=2)
      def _col(jc, carry2):
        rmins = list(carry2[:RB])
        ridxs = list(carry2[RB:])
        off = jc * L
        b0 = x2v[pl.ds(off, L)]
        b1 = x2v[pl.ds(M + off, L)]
        b2 = x2v[pl.ds(2 * M + off, L)]
        s2c = s2v[pl.ds(off, L)]
        jv = lanes + off
        cmin = cminv[pl.ds(off, L)]
        cidx = cidxv[pl.ds(off, L)]
        for r in range(RB):
          cross = (a0s[r] * b0 + a1s[r] * b1) + a2s[r] * b2
          d = (s1s[r] + s2c) - 2.0 * cross
          mr = d < rmins[r]
          rmins[r] = jnp.where(mr, d, rmins[r])
          ridxs[r] = jnp.where(mr, jv, ridxs[r])
          mc = d < cmin
          cmin = jnp.where(mc, d, cmin)
          cidx = jnp.where(mc, ivs[r], cidx)
        cminv[pl.ds(off, L)] = cmin
        cidxv[pl.ds(off, L)] = cidx
        return tuple(rmins + ridxs)

      res = _col
      for r in range(RB):
        rmin, ridx = res[r], res[RB + r]
        rs = jnp.min(rmin)
        ri = jnp.min(jnp.where(rmin == rs, ridx, jnp.int32(M)))
        lm = lanes == q * RB + r
        accm = jnp.where(lm, rs, accm)
        acci = jnp.where(lm, ri, acci)
      return accm, acci, g

    def _rowgrp(g, _):
      accm, acci, _g = lax.fori_loop(0, L // RB, _quad, (inf16, zero16, g))
      rminv[pl.ds(g * L, L)] = accm
      ridxv[pl.ds(g * L, L)] = acci
      return 0
    lax.fori_loop(0, chunk // L, _rowgrp, 0)

    pltpu.sync_copy(rminv, d1_hbm.at[b, pl.ds(row0, chunk)])
    pltpu.sync_copy(ridxv, i1_hbm.at[b, pl.ds(row0, chunk)])

    pltpu.sync_copy(cminv, shmin.at[pl.ds(s * M, M)])
    pltpu.sync_copy(cidxv, shidx.at[pl.ds(s * M, M)])
    plsc.subcore_barrier()

    @pl.when(ch == 0)
    def _merge():
      pltpu.sync_copy(shmin.at[pl.ds(s * M, wpb * M)], mmin)
      pltpu.sync_copy(shidx.at[pl.ds(s * M, wpb * M)], midx)

      @plsc.parallel_loop(0, M // L, unroll=4)
      def _mrg(jc):
        off = jc * L
        m = mmin[pl.ds(off, L)]
        ix = midx[pl.ds(off, L)]
        for k in range(1, wpb):
          mk = mmin[pl.ds(k * M + off, L)]
          ik = midx[pl.ds(k * M + off, L)]
          lt = mk < m
          m = jnp.where(lt, mk, m)
          ix = jnp.where(lt, ik, ix)
        cminv[pl.ds(off, L)] = m
        cidxv[pl.ds(off, L)] = ix
      pltpu.sync_copy(cminv, d2_hbm.at[b])
      pltpu.sync_copy(cidxv, i2_hbm.at[b])

  return _chamfer_sc


_chamfer_sc = _make_sc_kernel(SCB)


def _tc_body(x1_ref, x2_ref, d1_ref, d2_ref, i1_ref, i2_ref, cminp, cidxp):
  t = pl.program_id(1)
  nt = pl.num_programs(1)
  a = x1_ref[0]
  A0, A1, A2 = a[:, 0:1], a[:, 1:2], a[:, 2:3]
  s1 = (A0 * A0 + A1 * A1) + A2 * A2
  bb = x2_ref[0]
  B0, B1, B2 = bb[0:1, :], bb[1:2, :], bb[2:3, :]
  s2 = (B0 * B0 + B1 * B1) + B2 * B2
  ncross2 = lax.dot_general(-2.0 * a, bb, (((1,), (0,)), ((), ())),
                            preferred_element_type=jnp.float32)
  d = (s1 + s2) + ncross2

  jiota = lax.broadcasted_iota(jnp.int32, (TC_R, M), 1).astype(jnp.float32)
  rmin = jnp.min(d, axis=1, keepdims=True)
  ridx = jnp.min(jnp.where(d == rmin, jiota, float(M)), axis=1, keepdims=True)
  d1_ref[0] = rmin
  i1_ref[0] = ridx.astype(jnp.int32)

  riota = (lax.broadcasted_iota(jnp.int32, (TC_R, M), 0).astype(jnp.float32)
           + (t * TC_R).astype(jnp.float32))
  tcmin = jnp.min(d, axis=0, keepdims=True)
  tcidx = jnp.min(jnp.where(d == tcmin, riota, float(N)), axis=0, keepdims=True)

  @pl.when(t == 0)
  def _():
    cminp[...] = jnp.full((1, M), jnp.inf, jnp.float32)
    cidxp[...] = jnp.zeros((1, M), jnp.float32)

  upd = tcmin < cminp[...]
  cminp[...] = jnp.where(upd, tcmin, cminp[...])
  cidxp[...] = jnp.where(upd, tcidx, cidxp[...])

  @pl.when(t == nt - 1)
  def _():
    d2_ref[0] = cminp[...]
    i2_ref[0] = cidxp[...].astype(jnp.int32)


def _chamfer_tc(x1, x2t):
  nb = x1.shape[0]
  nt = N // TC_R
  out = pl.pallas_call(
      _tc_body,
      grid=(nb, nt),
      in_specs=[
          pl.BlockSpec((1, TC_R, 3), lambda b, t: (b, t, 0)),
          pl.BlockSpec((1, 3, M), lambda b, t: (b, 0, 0)),
      ],
      out_specs=[
          pl.BlockSpec((1, TC_R, 1), lambda b, t: (b, t, 0)),
          pl.BlockSpec((1, 1, M), lambda b, t: (b, 0, 0)),
          pl.BlockSpec((1, TC_R, 1), lambda b, t: (b, t, 0)),
          pl.BlockSpec((1, 1, M), lambda b, t: (b, 0, 0)),
      ],
      out_shape=[
          jax.ShapeDtypeStruct((nb, N, 1), jnp.float32),
          jax.ShapeDtypeStruct((nb, 1, M), jnp.float32),
          jax.ShapeDtypeStruct((nb, N, 1), jnp.int32),
          jax.ShapeDtypeStruct((nb, 1, M), jnp.int32),
      ],
      scratch_shapes=[
          pltpu.VMEM((1, M), jnp.float32),
          pltpu.VMEM((1, M), jnp.float32),
      ],
  )(x1, x2t)
  d1, d2, i1, i2 = out
  return (d1.reshape(nb, N), d2.reshape(nb, M),
          i1.reshape(nb, N), i2.reshape(nb, M))


@jax.jit
def kernel(input1, input2):
  x2t = jnp.swapaxes(input2, 1, 2)
  return _chamfer_tc(input1, x2t)


@jax.jit
def _kernel_hybrid(input1, input2):
  x2t = jnp.swapaxes(input2, 1, 2)
  x1f_sc = input1[:SCB].reshape(SCB, N * 3)
  x2t_sc = x2t[:SCB].reshape(SCB, 3 * M)
  sd1, sd2, si1, si2 = _chamfer_sc(x1f_sc, x2t_sc)
  td1, td2, ti1, ti2 = _chamfer_tc(input1[SCB:], x2t[SCB:])
  d1 = jnp.concatenate([sd1, td1], axis=0)
  d2 = jnp.concatenate([sd2, td2], axis=0)
  i1 = jnp.concatenate([si1, ti1], axis=0)
  i2 = jnp.concatenate([si2, ti2], axis=0)
  return d1, d2, i1, i2


# speedup vs baseline: 11.0544x; 1.1398x over previous
import functools

import jax
import jax.numpy as jnp
from jax import lax
from jax.experimental import pallas as pl
from jax.experimental.pallas import tpu as pltpu
from jax.experimental.pallas import tpu_sc as plsc

NC = 2
NS = 16
L = 16
B = 8
N = 2048
M = 2048

SCB = 2
TCB = B - SCB
TC_R = 2048

_mesh = plsc.VectorSubcoreMesh(core_axis_name="c", subcore_axis_name="s",
                               num_cores=NC, num_subcores=NS)


def _make_sc_kernel(nb):
  wpb = NC * NS // nb
  chunk = N // wpb

  @functools.partial(
      pl.kernel,
      out_type=(
          jax.ShapeDtypeStruct((nb, N), jnp.float32),
          jax.ShapeDtypeStruct((nb, M), jnp.float32),
          jax.ShapeDtypeStruct((nb, N), jnp.int32),
          jax.ShapeDtypeStruct((nb, M), jnp.int32),
      ),
      mesh=_mesh,
      compiler_params=pltpu.CompilerParams(needs_layout_passes=False),
      scratch_types=dict(
          x1v=pltpu.VMEM((chunk * 3 + L,), jnp.float32),
          x1r=pltpu.VMEM((chunk * 3 + L,), jnp.float32),
          x2v=pltpu.VMEM((3 * M,), jnp.float32),
          s2v=pltpu.VMEM((M,), jnp.float32),
          rminv=pltpu.VMEM((chunk,), jnp.float32),
          ridxv=pltpu.VMEM((chunk,), jnp.int32),
          cminv=pltpu.VMEM((M,), jnp.float32),
          cidxv=pltpu.VMEM((M,), jnp.int32),
          mmin=pltpu.VMEM((wpb * M,), jnp.float32),
          midx=pltpu.VMEM((wpb * M,), jnp.int32),
          shmin=pltpu.VMEM_SHARED((NS * M,), jnp.float32),
          shidx=pltpu.VMEM_SHARED((NS * M,), jnp.int32),
      ),
  )
  def _chamfer_sc(x1_hbm, x2_hbm, d1_hbm, d2_hbm, i1_hbm, i2_hbm,
                  x1v, x1r, x2v, s2v, rminv, ridxv, cminv, cidxv,
                  mmin, midx, shmin, shidx):
    c = lax.axis_index("c")
    s = lax.axis_index("s")
    wid = c * NS + s
    b = wid // wpb
    ch = wid % wpb
    row0 = ch * chunk

    pltpu.sync_copy(x1_hbm.at[b, pl.ds(row0 * 3, chunk * 3)],
                    x1v.at[pl.ds(0, chunk * 3)])
    pltpu.sync_copy(x2_hbm.at[b], x2v)

    lanes = lax.iota(jnp.int32, L)
    inf16 = jnp.full((L,), jnp.inf, jnp.float32)
    zero16 = jnp.zeros((L,), jnp.int32)

    def _bf16r(v):
      u = plsc.bitcast(v, jnp.uint32)
      u = (u + jnp.uint32(0x7FFF) + ((u >> jnp.uint32(16)) & jnp.uint32(1)))
      u = u & jnp.uint32(0xFFFF0000)
      return plsc.bitcast(u, jnp.float32)

    @plsc.parallel_loop(0, M // L, unroll=4)
    def _prep(jc):
      off = jc * L
      b0 = x2v[pl.ds(off, L)]
      b1 = x2v[pl.ds(M + off, L)]
      b2 = x2v[pl.ds(2 * M + off, L)]
      s2v[pl.ds(off, L)] = (b0 * b0 + b1 * b1) + b2 * b2
      x2v[pl.ds(off, L)] = _bf16r(b0)
      x2v[pl.ds(M + off, L)] = _bf16r(b1)
      x2v[pl.ds(2 * M + off, L)] = _bf16r(b2)
      cminv[pl.ds(off, L)] = inf16
      cidxv[pl.ds(off, L)] = zero16

    @plsc.parallel_loop(0, (chunk * 3 + L) // L, unroll=4)
    def _prep1(jc):
      off = jc * L
      x1r[pl.ds(off, L)] = _bf16r(x1v[pl.ds(off, L)])

    RB = 4

    def _quad(q, carry):
      accm, acci, g = carry
      i0 = g * L + q * RB
      s1s, a0s, a1s, a2s, ivs = [], [], [], [], []
      for r in range(RB):
        va = x1v[pl.ds(3 * (i0 + r), L)]
        A0 = jnp.full((L,), va[0], jnp.float32)
        A1 = jnp.full((L,), va[1], jnp.float32)
        A2 = jnp.full((L,), va[2], jnp.float32)
        s1s.append((A0 * A0 + A1 * A1) + A2 * A2)
        vb = x1r[pl.ds(3 * (i0 + r), L)]
        a0s.append(jnp.full((L,), vb[0], jnp.float32))
        a1s.append(jnp.full((L,), vb[1], jnp.float32))
        a2s.append(jnp.full((L,), vb[2], jnp.float32))
        ivs.append(jnp.full((L,), row0 + i0 + r, jnp.int32))

      init = tuple([inf16] * RB + [zero16] * RB)

      @plsc.parallel_loop(0, M // L, carry=init, unroll=2)
      def _col(jc, carry2):
        rmins = list(carry2[:RB])
        ridxs = list(carry2[RB:])
        off = jc * L
        b0 = x2v[pl.ds(off, L)]
        b1 = x2v[pl.ds(M + off, L)]
        b2 = x2v[pl.ds(2 * M + off, L)]
        s2c = s2v[pl.ds(off, L)]
        jv = lanes + off
        cmin = cminv[pl.ds(off, L)]
        cidx = cidxv[pl.ds(off, L)]
        for r in range(RB):
          cross = (a0s[r] * b0 + a1s[r] * b1) + a2s[r] * b2
          d = (s1s[r] + s2c) - 2.0 * cross
          mr = d < rmins[r]
          rmins[r] = jnp.where(mr, d, rmins[r])
          ridxs[r] = jnp.where(mr, jv, ridxs[r])
          mc = d < cmin
          cmin = jnp.where(mc, d, cmin)
          cidx = jnp.where(mc, ivs[r], cidx)
        cminv[pl.ds(off, L)] = cmin
        cidxv[pl.ds(off, L)] = cidx
        return tuple(rmins + ridxs)

      res = _col
      for r in range(RB):
        rmin, ridx = res[r], res[RB + r]
        rs = jnp.min(rmin)
        ri = jnp.min(jnp.where(rmin == rs, ridx, jnp.int32(M)))
        lm = lanes == q * RB + r
        accm = jnp.where(lm, rs, accm)
        acci = jnp.where(lm, ri, acci)
      return accm, acci, g

    def _rowgrp(g, _):
      accm, acci, _g = lax.fori_loop(0, L // RB, _quad, (inf16, zero16, g))
      rminv[pl.ds(g * L, L)] = accm
      ridxv[pl.ds(g * L, L)] = acci
      return 0
    lax.fori_loop(0, chunk // L, _rowgrp, 0)

    pltpu.sync_copy(rminv, d1_hbm.at[b, pl.ds(row0, chunk)])
    pltpu.sync_copy(ridxv, i1_hbm.at[b, pl.ds(row0, chunk)])

    pltpu.sync_copy(cminv, shmin.at[pl.ds(s * M, M)])
    pltpu.sync_copy(cidxv, shidx.at[pl.ds(s * M, M)])
    plsc.subcore_barrier()

    @pl.when(ch == 0)
    def _merge():
      pltpu.sync_copy(shmin.at[pl.ds(s * M, wpb * M)], mmin)
      pltpu.sync_copy(shidx.at[pl.ds(s * M, wpb * M)], midx)

      @plsc.parallel_loop(0, M // L, unroll=4)
      def _mrg(jc):
        off = jc * L
        m = mmin[pl.ds(off, L)]
        ix = midx[pl.ds(off, L)]
        for k in range(1, wpb):
          mk = mmin[pl.ds(k * M + off, L)]
          ik = midx[pl.ds(k * M + off, L)]
          lt = mk < m
          m = jnp.where(lt, mk, m)
          ix = jnp.where(lt, ik, ix)
        cminv[pl.ds(off, L)] = m
        cidxv[pl.ds(off, L)] = ix
      pltpu.sync_copy(cminv, d2_hbm.at[b])
      pltpu.sync_copy(cidxv, i2_hbm.at[b])

  return _chamfer_sc


_chamfer_sc = _make_sc_kernel(SCB)


def _tc_body(x1_ref, x2_ref, d1_ref, d2_ref, i1_ref, i2_ref, cminp, cidxp):
  t = pl.program_id(1)
  nt = pl.num_programs(1)
  a = x1_ref[0]
  A0, A1, A2 = a[:, 0:1], a[:, 1:2], a[:, 2:3]
  s1 = (A0 * A0 + A1 * A1) + A2 * A2
  bb = x2_ref[0]
  B0, B1, B2 = bb[0:1, :], bb[1:2, :], bb[2:3, :]
  s2 = (B0 * B0 + B1 * B1) + B2 * B2
  ncross2 = lax.dot_general(-2.0 * a, bb, (((1,), (0,)), ((), ())),
                            preferred_element_type=jnp.float32)
  d = (s1 + s2) + ncross2

  jiota = lax.broadcasted_iota(jnp.int32, (TC_R, M), 1).astype(jnp.float32)
  rmin = jnp.min(d, axis=1, keepdims=True)
  ridx = jnp.min(jnp.where(d == rmin, jiota, float(M)), axis=1, keepdims=True)
  d1_ref[0] = jnp.swapaxes(rmin, 0, 1)
  i1_ref[0] = jnp.swapaxes(ridx, 0, 1).astype(jnp.int32)

  riota = (lax.broadcasted_iota(jnp.int32, (TC_R, M), 0).astype(jnp.float32)
           + (t * TC_R).astype(jnp.float32))
  tcmin = jnp.min(d, axis=0, keepdims=True)
  tcidx = jnp.min(jnp.where(d == tcmin, riota, float(N)), axis=0, keepdims=True)

  @pl.when(t == 0)
  def _():
    cminp[...] = jnp.full((1, M), jnp.inf, jnp.float32)
    cidxp[...] = jnp.zeros((1, M), jnp.float32)

  upd = tcmin < cminp[...]
  cminp[...] = jnp.where(upd, tcmin, cminp[...])
  cidxp[...] = jnp.where(upd, tcidx, cidxp[...])

  @pl.when(t == nt - 1)
  def _():
    d2_ref[0] = cminp[...]
    i2_ref[0] = cidxp[...].astype(jnp.int32)


def _chamfer_tc(x1, x2t):
  nb = x1.shape[0]
  nt = N // TC_R
  out = pl.pallas_call(
      _tc_body,
      grid=(nb, nt),
      in_specs=[
          pl.BlockSpec((1, TC_R, 3), lambda b, t: (b, t, 0)),
          pl.BlockSpec((1, 3, M), lambda b, t: (b, 0, 0)),
      ],
      out_specs=[
          pl.BlockSpec((1, 1, TC_R), lambda b, t: (b, 0, t)),
          pl.BlockSpec((1, 1, M), lambda b, t: (b, 0, 0)),
          pl.BlockSpec((1, 1, TC_R), lambda b, t: (b, 0, t)),
          pl.BlockSpec((1, 1, M), lambda b, t: (b, 0, 0)),
      ],
      out_shape=[
          jax.ShapeDtypeStruct((nb, 1, N), jnp.float32),
          jax.ShapeDtypeStruct((nb, 1, M), jnp.float32),
          jax.ShapeDtypeStruct((nb, 1, N), jnp.int32),
          jax.ShapeDtypeStruct((nb, 1, M), jnp.int32),
      ],
      scratch_shapes=[
          pltpu.VMEM((1, M), jnp.float32),
          pltpu.VMEM((1, M), jnp.float32),
      ],
  )(x1, x2t)
  d1, d2, i1, i2 = out
  return (d1.reshape(nb, N), d2.reshape(nb, M),
          i1.reshape(nb, N), i2.reshape(nb, M))


@jax.jit
def kernel(input1, input2):
  x2t = jnp.swapaxes(input2, 1, 2)
  return _chamfer_tc(input1, x2t)


@jax.jit
def _kernel_hybrid(input1, input2):
  x2t = jnp.swapaxes(input2, 1, 2)
  x1f_sc = input1[:SCB].reshape(SCB, N * 3)
  x2t_sc = x2t[:SCB].reshape(SCB, 3 * M)
  sd1, sd2, si1, si2 = _chamfer_sc(x1f_sc, x2t_sc)
  td1, td2, ti1, ti2 = _chamfer_tc(input1[SCB:], x2t[SCB:])
  d1 = jnp.concatenate([sd1, td1], axis=0)
  d2 = jnp.concatenate([sd2, td2], axis=0)
  i1 = jnp.concatenate([si1, ti1], axis=0)
  i2 = jnp.concatenate([si2, ti2], axis=0)
  return d1, d2, i1, i2
